# Initial kernel scaffold; baseline (speedup 1.0000x reference)
import jax
import jax.numpy as jnp
from jax.experimental import pallas as pl


def kernel(x, edge_index, batch, W_rel1, W_root1, b1, W_rel2, W_root2, b2, ln_g, ln_b, bn_g, bn_b, W_l1, b_l1, W_l2, b_l2, W_l3, b_l3):
    raise NotImplementedError("write your pallas kernel here")


# speedup vs baseline: 3.8002x; 3.8002x over previous
import functools

import jax
import jax.numpy as jnp
from jax import lax
from jax.experimental import pallas as pl
from jax.experimental.pallas import tpu as pltpu
from jax.experimental.pallas import tpu_sc as plsc

N = 10000
E = 160000
D = 256
HD = 128
NG = 64
NCLS = 2
NC, NS = 2, 16
EPS = 1e-5
NEG = -3.4e38

EK = 80
EPT = E // NS
NCHUNK = EPT // EK
RPT = 640
RPT_LAST = N - (NS - 1) * RPT
GPT = 8

BLK = 1000

_GDN = lax.GatherDimensionNumbers(
    offset_dims=(), collapsed_slice_dims=(0,), start_index_map=(0,))


def _splat_lane(vec, r):
    idx = jnp.full((16, 1), r, jnp.int32)
    return lax.gather(vec, idx, dimension_numbers=_GDN, slice_sizes=(1,),
                      mode=lax.GatherScatterMode.PROMISE_IN_BOUNDS)


def _mm_body(h_ref, w_ref, b_ref, yrel_ref, z_ref):
    y = jnp.dot(h_ref[...], w_ref[...], preferred_element_type=jnp.float32)
    yrel_ref[0] = y[:, :HD]
    yrel_ref[1] = y[:, HD:D]
    z_ref[...] = y[:, D:] + b_ref[...]


def _matmul(h, wcat, b2d):
    return pl.pallas_call(
        _mm_body,
        grid=(N // BLK,),
        in_specs=[
            pl.BlockSpec((BLK, D), lambda i: (i, 0)),
            pl.BlockSpec((D, 2 * D), lambda i: (0, 0)),
            pl.BlockSpec((1, D), lambda i: (0, 0)),
        ],
        out_specs=[
            pl.BlockSpec((NC, BLK, HD), lambda i: (0, i, 0)),
            pl.BlockSpec((BLK, D), lambda i: (i, 0)),
        ],
        out_shape=[
            jax.ShapeDtypeStruct((NC, N, HD), jnp.float32),
            jax.ShapeDtypeStruct((N, D), jnp.float32),
        ],
    )(h, wcat, b2d)


def _ln_body(agg_ref, z_ref, g_ref, b_ref, h_ref, h2_ref):
    a = jnp.concatenate([agg_ref[0], agg_ref[1]], axis=1) + z_ref[...]
    m = jnp.mean(a, axis=1, keepdims=True)
    v = jnp.mean((a - m) ** 2, axis=1, keepdims=True)
    hn = (a - m) * lax.rsqrt(v + EPS) * g_ref[...] + b_ref[...]
    h_ref[...] = hn
    h2_ref[0] = hn[:, :HD]
    h2_ref[1] = hn[:, HD:]


def _layernorm(agg, z, g2d, b2d):
    return pl.pallas_call(
        _ln_body,
        grid=(N // BLK,),
        in_specs=[
            pl.BlockSpec((NC, BLK, HD), lambda i: (0, i, 0)),
            pl.BlockSpec((BLK, D), lambda i: (i, 0)),
            pl.BlockSpec((1, D), lambda i: (0, 0)),
            pl.BlockSpec((1, D), lambda i: (0, 0)),
        ],
        out_specs=[
            pl.BlockSpec((BLK, D), lambda i: (i, 0)),
            pl.BlockSpec((NC, BLK, HD), lambda i: (0, i, 0)),
        ],
        out_shape=[
            jax.ShapeDtypeStruct((N, D), jnp.float32),
            jax.ShapeDtypeStruct((NC, N, HD), jnp.float32),
        ],
    )(agg, z, g2d, b2d)


_SC_MESH = plsc.VectorSubcoreMesh(core_axis_name="c", subcore_axis_name="s")


@functools.partial(
    pl.kernel,
    out_type=jax.ShapeDtypeStruct((NC, N, HD), jnp.float32),
    mesh=_SC_MESH,
    scratch_types=[
        pltpu.VMEM((NCHUNK, EK), jnp.int32),
        pltpu.VMEM((NCHUNK, EK), jnp.int32),
        pltpu.VMEM((EK, HD), jnp.float32),
        pltpu.VMEM_SHARED((N, HD), jnp.float32),
        pltpu.SemaphoreType.DMA,
    ],
)
def _edge_kernel(ytab, src3, dst3, zeros, agg_out, src_v, dst_v, rows_v, acc, sem):
    c = lax.axis_index("c")
    s = lax.axis_index("s")
    @pl.when(s < NS - 1)
    def _():
        pltpu.sync_copy(zeros.at[pl.ds(s * RPT, RPT)], acc.at[pl.ds(s * RPT, RPT)])

    @pl.when(s == NS - 1)
    def _():
        pltpu.sync_copy(zeros.at[pl.ds(s * RPT, RPT_LAST)],
                        acc.at[pl.ds(s * RPT, RPT_LAST)])

    pltpu.sync_copy(src3.at[s], src_v)
    pltpu.sync_copy(dst3.at[s], dst_v)
    off = c * N

    def _add_off(i, _):
        for j in range(EK // 16):
            src_v[i, pl.ds(j * 16, 16)] = src_v[i, pl.ds(j * 16, 16)] + off
        return 0

    lax.fori_loop(0, NCHUNK, _add_off, 0)
    plsc.subcore_barrier()

    def _chunk(i, _):
        pltpu.async_copy(ytab.at[src_v.at[i]], rows_v, sem).wait()
        pltpu.sync_copy(rows_v, acc.at[dst_v.at[i]], add=True)
        return 0

    lax.fori_loop(0, NCHUNK, _chunk, 0)
    plsc.subcore_barrier()

    @pl.when(s < NS - 1)
    def _():
        pltpu.sync_copy(acc.at[pl.ds(s * RPT, RPT)],
                        agg_out.at[c, pl.ds(s * RPT, RPT)])

    @pl.when(s == NS - 1)
    def _():
        pltpu.sync_copy(acc.at[pl.ds(s * RPT, RPT_LAST)],
                        agg_out.at[c, pl.ds(s * RPT, RPT_LAST)])


@functools.partial(
    pl.kernel,
    out_type=(
        jax.ShapeDtypeStruct((NC, NG * HD), jnp.float32),
        jax.ShapeDtypeStruct((NC, NG * HD), jnp.float32),
    ),
    mesh=_SC_MESH,
    compiler_params=pltpu.CompilerParams(needs_layout_passes=False),
    scratch_types=[
        pltpu.VMEM((RPT, HD), jnp.float32),
        pltpu.VMEM((RPT,), jnp.int32),
        pltpu.VMEM((NG * HD,), jnp.float32),
        pltpu.VMEM((NG * HD,), jnp.float32),
        pltpu.VMEM((GPT * HD,), jnp.float32),
        pltpu.VMEM((GPT * HD,), jnp.float32),
        pltpu.VMEM_SHARED((NS, NG * HD), jnp.float32),
    ],
)
def _pool_kernel(h2, batch, psum, pmax, rows_v, bid_v, sum_v, max_v, tmp_v, res_v, stage):
    c = lax.axis_index("c")
    s = lax.axis_index("s")
    base = s * RPT
    iota = lax.iota(jnp.int32, 16)
    z16 = jnp.zeros((16,), jnp.float32)
    neg16 = jnp.full((16,), NEG, jnp.float32)

    @pl.when(s < NS - 1)
    def _():
        pltpu.sync_copy(h2.at[c, pl.ds(base, RPT)], rows_v)
        pltpu.sync_copy(batch.at[pl.ds(base, RPT)], bid_v)

    @pl.when(s == NS - 1)
    def _():
        pltpu.sync_copy(h2.at[c, pl.ds(base, RPT_LAST)], rows_v.at[pl.ds(0, RPT_LAST)])
        pltpu.sync_copy(batch.at[pl.ds(base, RPT_LAST)], bid_v.at[pl.ds(0, RPT_LAST)])

    def _init(i, _):
        sum_v[pl.ds(i * 16, 16)] = z16
        max_v[pl.ds(i * 16, 16)] = neg16
        return 0

    lax.fori_loop(0, NG * HD // 16, _init, 0)

    ngroups = jnp.where(s < NS - 1, RPT // 16, RPT_LAST // 16)

    def _group(gi, _):
        gbase = gi * 16
        bvec = bid_v[pl.ds(gbase, 16)]
        for r in range(16):
            rowbase = _splat_lane(bvec, r) * HD
            for j in range(HD // 16):
                fi = rowbase + (j * 16 + iota)
                v = rows_v[gbase + r, pl.ds(j * 16, 16)]
                cs = plsc.load_gather(sum_v, [fi])
                plsc.store_scatter(sum_v, [fi], cs + v)
                cm = plsc.load_gather(max_v, [fi])
                plsc.store_scatter(max_v, [fi], jnp.maximum(cm, v))
        return 0

    lax.fori_loop(0, ngroups, _group, 0)

    seg = GPT * HD

    pltpu.sync_copy(sum_v, stage.at[s])
    plsc.subcore_barrier()

    @pl.when(s < NG // GPT)
    def _():
        def _zero(j, _):
            res_v[pl.ds(j * 16, 16)] = z16
            return 0

        lax.fori_loop(0, seg // 16, _zero, 0)

        def _msum(t, _):
            pltpu.sync_copy(stage.at[t, pl.ds(seg * s, seg)], tmp_v)
            for j in range(seg // 16):
                res_v[pl.ds(j * 16, 16)] = (
                    res_v[pl.ds(j * 16, 16)] + tmp_v[pl.ds(j * 16, 16)])
            return 0

        lax.fori_loop(0, NS, _msum, 0)
        pltpu.sync_copy(res_v, psum.at[c, pl.ds(seg * s, seg)])

    plsc.subcore_barrier()
    pltpu.sync_copy(max_v, stage.at[s])
    plsc.subcore_barrier()

    @pl.when(s < NG // GPT)
    def _():
        def _zneg(j, _):
            res_v[pl.ds(j * 16, 16)] = neg16
            return 0

        lax.fori_loop(0, seg // 16, _zneg, 0)

        def _mmax(t, _):
            pltpu.sync_copy(stage.at[t, pl.ds(seg * s, seg)], tmp_v)
            for j in range(seg // 16):
                res_v[pl.ds(j * 16, 16)] = jnp.maximum(
                    res_v[pl.ds(j * 16, 16)], tmp_v[pl.ds(j * 16, 16)])
            return 0

        lax.fori_loop(0, NS, _mmax, 0)
        pltpu.sync_copy(res_v, pmax.at[c, pl.ds(seg * s, seg)])


def _head_body(ps1, px1, ps2, px2, ps3, px3, bt, bng, bnb,
               w1, bl1, w2, bl2, w3, bl3, out_ref):
    gids = lax.broadcasted_iota(jnp.int32, (NG, 1), 0)
    cnt = jnp.sum((bt[...] == gids).astype(jnp.float32), axis=1, keepdims=True)
    cnt_c = jnp.maximum(cnt, 1.0)
    pieces = []
    for ps, px in ((ps1, px1), (ps2, px2), (ps3, px3)):
        sm = jnp.concatenate([ps[0], ps[1]], axis=1)
        mx = jnp.concatenate([px[0], px[1]], axis=1)
        pieces += [sm / cnt_c, jnp.where(cnt > 0.0, mx, 0.0), sm]
    hk = jnp.concatenate(pieces, axis=1)
    bm = jnp.mean(hk, axis=0, keepdims=True)
    bv = jnp.mean((hk - bm) ** 2, axis=0, keepdims=True)
    xn = (hk - bm) * lax.rsqrt(bv + EPS) * bng[...] + bnb[...]
    x1 = jnp.maximum(jnp.dot(xn, w1[...], preferred_element_type=jnp.float32) + bl1[...], 0.0)
    x2 = jnp.maximum(jnp.dot(x1, w2[...], preferred_element_type=jnp.float32) + bl2[...], 0.0)
    lg = jnp.dot(x2, w3[...], preferred_element_type=jnp.float32) + bl3[...]
    valid = lax.broadcasted_iota(jnp.int32, (NG, HD), 1) < NCLS
    mxl = jnp.max(jnp.where(valid, lg, NEG), axis=1, keepdims=True)
    ex = jnp.where(valid, jnp.exp(lg - mxl), 0.0)
    lse = jnp.log(jnp.sum(ex, axis=1, keepdims=True)) + mxl
    out_ref[...] = lg - lse


def _head(pools, batch2d, bng, bnb, w1, bl1, w2, bl2, w3p, bl3p):
    return pl.pallas_call(
        _head_body,
        out_shape=jax.ShapeDtypeStruct((NG, HD), jnp.float32),
    )(pools[0], pools[1], pools[2], pools[3], pools[4], pools[5],
      batch2d, bng, bnb, w1, bl1, w2, bl2, w3p, bl3p)


def kernel(x, edge_index, batch, W_rel1, W_root1, b1, W_rel2, W_root2, b2,
           ln_g, ln_b, bn_g, bn_b, W_l1, b_l1, W_l2, b_l2, W_l3, b_l3):
    src3 = edge_index[0].reshape(NS, NCHUNK, EK)
    dst3 = edge_index[1].reshape(NS, NCHUNK, EK)
    zeros_nh = jnp.zeros((N, HD), jnp.float32)
    wcat1 = jnp.concatenate([W_rel1, W_root1], axis=1)
    wcat2 = jnp.concatenate([W_rel2, W_root2], axis=1)
    b1_2d = b1.reshape(1, D)
    b2_2d = b2.reshape(1, D)
    lng = ln_g.reshape(1, D)
    lnb = ln_b.reshape(1, D)

    h = x
    pools = []
    for k in range(3):
        wcat, bb = (wcat1, b1_2d) if k == 0 else (wcat2, b2_2d)
        yrel, z = _matmul(h, wcat, bb)
        agg = _edge_kernel(yrel.reshape(NC * N, HD), src3, dst3, zeros_nh)
        h, h2 = _layernorm(agg, z, lng, lnb)
        ps, px = _pool_kernel(h2, batch)
        pools += [ps.reshape(NC, NG, HD), px.reshape(NC, NG, HD)]

    w3p = jnp.pad(W_l3, ((0, 0), (0, HD - NCLS)))
    bl3p = jnp.pad(b_l3, (0, HD - NCLS)).reshape(1, HD)
    out128 = _head(pools, batch.reshape(1, N), bn_g.reshape(1, -1),
                   bn_b.reshape(1, -1), W_l1, b_l1.reshape(1, -1),
                   W_l2, b_l2.reshape(1, -1), w3p, bl3p)
    return out128[:, :NCLS]


# speedup vs baseline: 4.8386x; 1.2732x over previous
import functools

import jax
import jax.numpy as jnp
from jax import lax
from jax.experimental import pallas as pl
from jax.experimental.pallas import tpu as pltpu
from jax.experimental.pallas import tpu_sc as plsc

N = 10000
E = 160000
D = 256
HD = 128
NG = 64
NCLS = 2
NC, NS = 2, 16
EPS = 1e-5
NEG = -3.4e38

EK = 80
EPT = E // NS
NCHUNK = EPT // EK
IBLK = 25
NBLK = NCHUNK // IBLK
RPT = 640
RPT_LAST = N - (NS - 1) * RPT
GPT = 8

BLK = 1000

_GDN = lax.GatherDimensionNumbers(
    offset_dims=(), collapsed_slice_dims=(0,), start_index_map=(0,))


def _splat_lane(vec, r):
    idx = jnp.full((16, 1), r, jnp.int32)
    return lax.gather(vec, idx, dimension_numbers=_GDN, slice_sizes=(1,),
                      mode=lax.GatherScatterMode.PROMISE_IN_BOUNDS)


def _mm_body(h_ref, w_ref, b_ref, yrel_ref, z_ref):
    y = jnp.dot(h_ref[...], w_ref[...], preferred_element_type=jnp.float32)
    yrel_ref[0] = y[:, :HD]
    yrel_ref[1] = y[:, HD:D]
    z_ref[...] = y[:, D:] + b_ref[...]


def _matmul(h, wcat, b2d):
    return pl.pallas_call(
        _mm_body,
        grid=(N // BLK,),
        in_specs=[
            pl.BlockSpec((BLK, D), lambda i: (i, 0)),
            pl.BlockSpec((D, 2 * D), lambda i: (0, 0)),
            pl.BlockSpec((1, D), lambda i: (0, 0)),
        ],
        out_specs=[
            pl.BlockSpec((NC, BLK, HD), lambda i: (0, i, 0)),
            pl.BlockSpec((BLK, D), lambda i: (i, 0)),
        ],
        out_shape=[
            jax.ShapeDtypeStruct((NC, N, HD), jnp.float32),
            jax.ShapeDtypeStruct((N, D), jnp.float32),
        ],
    )(h, wcat, b2d)


def _ln_body(agg_ref, z_ref, g_ref, b_ref, h_ref, h2_ref):
    a = jnp.concatenate([agg_ref[0], agg_ref[1]], axis=1) + z_ref[...]
    m = jnp.mean(a, axis=1, keepdims=True)
    v = jnp.mean((a - m) ** 2, axis=1, keepdims=True)
    hn = (a - m) * lax.rsqrt(v + EPS) * g_ref[...] + b_ref[...]
    h_ref[...] = hn
    h2_ref[0] = hn[:, :HD]
    h2_ref[1] = hn[:, HD:]


def _layernorm(agg, z, g2d, b2d):
    return pl.pallas_call(
        _ln_body,
        grid=(N // BLK,),
        in_specs=[
            pl.BlockSpec((NC, BLK, HD), lambda i: (0, i, 0)),
            pl.BlockSpec((BLK, D), lambda i: (i, 0)),
            pl.BlockSpec((1, D), lambda i: (0, 0)),
            pl.BlockSpec((1, D), lambda i: (0, 0)),
        ],
        out_specs=[
            pl.BlockSpec((BLK, D), lambda i: (i, 0)),
            pl.BlockSpec((NC, BLK, HD), lambda i: (0, i, 0)),
        ],
        out_shape=[
            jax.ShapeDtypeStruct((N, D), jnp.float32),
            jax.ShapeDtypeStruct((NC, N, HD), jnp.float32),
        ],
    )(agg, z, g2d, b2d)


_SC_MESH = plsc.VectorSubcoreMesh(core_axis_name="c", subcore_axis_name="s")


@functools.partial(
    pl.kernel,
    out_type=jax.ShapeDtypeStruct((NC, N, HD), jnp.float32),
    mesh=_SC_MESH,
    scratch_types=[
        pltpu.VMEM((IBLK, EK), jnp.int32),
        pltpu.VMEM((IBLK, EK), jnp.int32),
        pltpu.VMEM((2, EK, HD), jnp.float32),
        pltpu.VMEM_SHARED((N, HD), jnp.float32),
        pltpu.SemaphoreType.DMA((2,)),
    ],
)
def _edge_kernel(ytab, src4, dst4, zeros, agg_out, src_v, dst_v, rows_v, acc, sem):
    c = lax.axis_index("c")
    s = lax.axis_index("s")
    @pl.when(s < NS - 1)
    def _():
        pltpu.sync_copy(zeros.at[pl.ds(s * RPT, RPT)], acc.at[pl.ds(s * RPT, RPT)])

    @pl.when(s == NS - 1)
    def _():
        pltpu.sync_copy(zeros.at[pl.ds(s * RPT, RPT_LAST)],
                        acc.at[pl.ds(s * RPT, RPT_LAST)])

    off = c * N
    plsc.subcore_barrier()

    def _block(blk, _):
        pltpu.sync_copy(src4.at[s, blk], src_v)
        pltpu.sync_copy(dst4.at[s, blk], dst_v)

        def _add_off(i, _):
            for j in range(EK // 16):
                src_v[i, pl.ds(j * 16, 16)] = src_v[i, pl.ds(j * 16, 16)] + off
            return 0

        lax.fori_loop(0, IBLK, _add_off, 0)
        pltpu.async_copy(ytab.at[src_v.at[0]], rows_v.at[0], sem.at[0])

        def _chunk(i, _):
            b = lax.rem(i, 2)
            nb = 1 - b
            pltpu.make_async_copy(ytab.at[src_v.at[i]], rows_v.at[b], sem.at[b]).wait()

            @pl.when(i + 1 < IBLK)
            def _():
                pltpu.async_copy(ytab.at[src_v.at[i + 1]], rows_v.at[nb], sem.at[nb])

            pltpu.sync_copy(rows_v.at[b], acc.at[dst_v.at[i]], add=True)
            return 0

        lax.fori_loop(0, IBLK, _chunk, 0)
        return 0

    lax.fori_loop(0, NBLK, _block, 0)
    plsc.subcore_barrier()

    @pl.when(s < NS - 1)
    def _():
        pltpu.sync_copy(acc.at[pl.ds(s * RPT, RPT)],
                        agg_out.at[c, pl.ds(s * RPT, RPT)])

    @pl.when(s == NS - 1)
    def _():
        pltpu.sync_copy(acc.at[pl.ds(s * RPT, RPT_LAST)],
                        agg_out.at[c, pl.ds(s * RPT, RPT_LAST)])


@functools.partial(
    pl.kernel,
    out_type=(
        jax.ShapeDtypeStruct((NC, NG * HD), jnp.float32),
        jax.ShapeDtypeStruct((NC, NG * HD), jnp.float32),
    ),
    mesh=_SC_MESH,
    compiler_params=pltpu.CompilerParams(needs_layout_passes=False),
    scratch_types=[
        pltpu.VMEM((RPT // 2, HD), jnp.float32),
        pltpu.VMEM((RPT,), jnp.int32),
        pltpu.VMEM((NG * HD,), jnp.float32),
        pltpu.VMEM((NG * HD,), jnp.float32),
        pltpu.VMEM((NS, GPT * HD), jnp.float32),
        pltpu.VMEM((GPT * HD,), jnp.float32),
        pltpu.VMEM_SHARED((NS, NG * HD), jnp.float32),
    ],
)
def _pool_kernel(h2, batch, psum, pmax, rows_v, bid_v, sum_v, max_v, tmp_v, res_v, stage):
    c = lax.axis_index("c")
    s = lax.axis_index("s")
    base = s * RPT
    iota = lax.iota(jnp.int32, 16)
    z16 = jnp.zeros((16,), jnp.float32)
    neg16 = jnp.full((16,), NEG, jnp.float32)

    @pl.when(s < NS - 1)
    def _():
        pltpu.sync_copy(batch.at[pl.ds(base, RPT)], bid_v)

    @pl.when(s == NS - 1)
    def _():
        pltpu.sync_copy(batch.at[pl.ds(base, RPT_LAST)], bid_v.at[pl.ds(0, RPT_LAST)])

    def _init(i, _):
        sum_v[pl.ds(i * 16, 16)] = z16
        max_v[pl.ds(i * 16, 16)] = neg16
        return 0

    lax.fori_loop(0, NG * HD // 16, _init, 0)

    def _group_at(poff, gi):
        gbase = gi * 16
        bvec = bid_v[pl.ds(poff + gbase, 16)]
        same = jnp.max(bvec) == jnp.min(bvec)

        @pl.when(same)
        def _():
            rowbase = _splat_lane(bvec, 0) * HD
            for j in range(HD // 16):
                acc_s = rows_v[gbase, pl.ds(j * 16, 16)]
                acc_m = acc_s
                for r in range(1, 16):
                    v = rows_v[gbase + r, pl.ds(j * 16, 16)]
                    acc_s = acc_s + v
                    acc_m = jnp.maximum(acc_m, v)
                fi = rowbase + (j * 16 + iota)
                cs = plsc.load_gather(sum_v, [fi])
                plsc.store_scatter(sum_v, [fi], cs + acc_s)
                cm = plsc.load_gather(max_v, [fi])
                plsc.store_scatter(max_v, [fi], jnp.maximum(cm, acc_m))

        @pl.when(jnp.logical_not(same))
        def _():
            for r in range(16):
                rowbase = _splat_lane(bvec, r) * HD
                for j in range(HD // 16):
                    fi = rowbase + (j * 16 + iota)
                    v = rows_v[gbase + r, pl.ds(j * 16, 16)]
                    cs = plsc.load_gather(sum_v, [fi])
                    plsc.store_scatter(sum_v, [fi], cs + v)
                    cm = plsc.load_gather(max_v, [fi])
                    plsc.store_scatter(max_v, [fi], jnp.maximum(cm, v))

    RPB = RPT // 2

    pltpu.sync_copy(h2.at[c, pl.ds(base, RPB)], rows_v)

    def _g0(gi, _):
        _group_at(0, gi)
        return 0

    lax.fori_loop(0, RPB // 16, _g0, 0)

    @pl.when(s < NS - 1)
    def _():
        pltpu.sync_copy(h2.at[c, pl.ds(base + RPB, RPB)], rows_v)

    @pl.when(s == NS - 1)
    def _():
        pltpu.sync_copy(h2.at[c, pl.ds(base + RPB, RPT_LAST - RPB)],
                        rows_v.at[pl.ds(0, RPT_LAST - RPB)])

    ng1 = jnp.where(s < NS - 1, RPB // 16, (RPT_LAST - RPB) // 16)

    def _g1(gi, _):
        _group_at(RPB, gi)
        return 0

    lax.fori_loop(0, ng1, _g1, 0)

    seg = GPT * HD

    pltpu.sync_copy(sum_v, stage.at[s])
    plsc.subcore_barrier()

    @pl.when(s < NG // GPT)
    def _():
        pltpu.sync_copy(stage.at[pl.ds(0, NS), pl.ds(seg * s, seg)], tmp_v)

        def _zero(j, _):
            res_v[pl.ds(j * 16, 16)] = z16
            return 0

        lax.fori_loop(0, seg // 16, _zero, 0)

        def _msum(t, _):
            for j in range(seg // 16):
                res_v[pl.ds(j * 16, 16)] = (
                    res_v[pl.ds(j * 16, 16)] + tmp_v[t, pl.ds(j * 16, 16)])
            return 0

        lax.fori_loop(0, NS, _msum, 0)
        pltpu.sync_copy(res_v, psum.at[c, pl.ds(seg * s, seg)])

    plsc.subcore_barrier()
    pltpu.sync_copy(max_v, stage.at[s])
    plsc.subcore_barrier()

    @pl.when(s < NG // GPT)
    def _():
        pltpu.sync_copy(stage.at[pl.ds(0, NS), pl.ds(seg * s, seg)], tmp_v)

        def _zneg(j, _):
            res_v[pl.ds(j * 16, 16)] = neg16
            return 0

        lax.fori_loop(0, seg // 16, _zneg, 0)

        def _mmax(t, _):
            for j in range(seg // 16):
                res_v[pl.ds(j * 16, 16)] = jnp.maximum(
                    res_v[pl.ds(j * 16, 16)], tmp_v[t, pl.ds(j * 16, 16)])
            return 0

        lax.fori_loop(0, NS, _mmax, 0)
        pltpu.sync_copy(res_v, pmax.at[c, pl.ds(seg * s, seg)])


def _head_body(ps1, px1, ps2, px2, ps3, px3, bt, bng, bnb,
               w1, bl1, w2, bl2, w3, bl3, out_ref):
    gids = lax.broadcasted_iota(jnp.int32, (NG, 1), 0)
    cnt = jnp.sum((bt[...] == gids).astype(jnp.float32), axis=1, keepdims=True)
    cnt_c = jnp.maximum(cnt, 1.0)
    pieces = []
    for ps, px in ((ps1, px1), (ps2, px2), (ps3, px3)):
        sm = jnp.concatenate([ps[0], ps[1]], axis=1)
        mx = jnp.concatenate([px[0], px[1]], axis=1)
        pieces += [sm / cnt_c, jnp.where(cnt > 0.0, mx, 0.0), sm]
    hk = jnp.concatenate(pieces, axis=1)
    bm = jnp.mean(hk, axis=0, keepdims=True)
    bv = jnp.mean((hk - bm) ** 2, axis=0, keepdims=True)
    xn = (hk - bm) * lax.rsqrt(bv + EPS) * bng[...] + bnb[...]
    x1 = jnp.maximum(jnp.dot(xn, w1[...], preferred_element_type=jnp.float32) + bl1[...], 0.0)
    x2 = jnp.maximum(jnp.dot(x1, w2[...], preferred_element_type=jnp.float32) + bl2[...], 0.0)
    lg = jnp.dot(x2, w3[...], preferred_element_type=jnp.float32) + bl3[...]
    valid = lax.broadcasted_iota(jnp.int32, (NG, HD), 1) < NCLS
    mxl = jnp.max(jnp.where(valid, lg, NEG), axis=1, keepdims=True)
    ex = jnp.where(valid, jnp.exp(lg - mxl), 0.0)
    lse = jnp.log(jnp.sum(ex, axis=1, keepdims=True)) + mxl
    out_ref[...] = lg - lse


def _head(pools, batch2d, bng, bnb, w1, bl1, w2, bl2, w3p, bl3p):
    return pl.pallas_call(
        _head_body,
        out_shape=jax.ShapeDtypeStruct((NG, HD), jnp.float32),
    )(pools[0], pools[1], pools[2], pools[3], pools[4], pools[5],
      batch2d, bng, bnb, w1, bl1, w2, bl2, w3p, bl3p)


def kernel(x, edge_index, batch, W_rel1, W_root1, b1, W_rel2, W_root2, b2,
           ln_g, ln_b, bn_g, bn_b, W_l1, b_l1, W_l2, b_l2, W_l3, b_l3):
    src4 = edge_index[0].reshape(NS, NBLK, IBLK, EK)
    dst4 = edge_index[1].reshape(NS, NBLK, IBLK, EK)
    zeros_nh = jnp.zeros((N, HD), jnp.float32)
    wcat1 = jnp.concatenate([W_rel1, W_root1], axis=1)
    wcat2 = jnp.concatenate([W_rel2, W_root2], axis=1)
    b1_2d = b1.reshape(1, D)
    b2_2d = b2.reshape(1, D)
    lng = ln_g.reshape(1, D)
    lnb = ln_b.reshape(1, D)

    h = x
    pools = []
    for k in range(3):
        wcat, bb = (wcat1, b1_2d) if k == 0 else (wcat2, b2_2d)
        yrel, z = _matmul(h, wcat, bb)
        agg = _edge_kernel(yrel.reshape(NC * N, HD), src4, dst4, zeros_nh)
        h, h2 = _layernorm(agg, z, lng, lnb)
        ps, px = _pool_kernel(h2, batch)
        pools += [ps.reshape(NC, NG, HD), px.reshape(NC, NG, HD)]

    w3p = jnp.pad(W_l3, ((0, 0), (0, HD - NCLS)))
    bl3p = jnp.pad(b_l3, (0, HD - NCLS)).reshape(1, HD)
    out128 = _head(pools, batch.reshape(1, N), bn_g.reshape(1, -1),
                   bn_b.reshape(1, -1), W_l1, b_l1.reshape(1, -1),
                   W_l2, b_l2.reshape(1, -1), w3p, bl3p)
    return out128[:, :NCLS]


# speedup vs baseline: 4.8726x; 1.0070x over previous
import functools

import jax
import jax.numpy as jnp
from jax import lax
from jax.experimental import pallas as pl
from jax.experimental.pallas import tpu as pltpu
from jax.experimental.pallas import tpu_sc as plsc

N = 10000
E = 160000
D = 256
HD = 128
NG = 64
NCLS = 2
NC, NS = 2, 16
EPS = 1e-5
NEG = -3.4e38

EK = 80
EPT = E // NS
NCHUNK = EPT // EK
IBLK = 25
NBLK = NCHUNK // IBLK
RPT = 640
RPT_LAST = N - (NS - 1) * RPT
GPT = 8

BLK = 1000

_GDN = lax.GatherDimensionNumbers(
    offset_dims=(), collapsed_slice_dims=(0,), start_index_map=(0,))


def _splat_lane(vec, r):
    idx = jnp.full((16, 1), r, jnp.int32)
    return lax.gather(vec, idx, dimension_numbers=_GDN, slice_sizes=(1,),
                      mode=lax.GatherScatterMode.PROMISE_IN_BOUNDS)


def _mm_body(h_ref, w_ref, b_ref, yrel_ref, z_ref):
    y = jnp.dot(h_ref[...], w_ref[...], preferred_element_type=jnp.float32)
    yrel_ref[0] = y[:, :HD]
    yrel_ref[1] = y[:, HD:D]
    z_ref[...] = y[:, D:] + b_ref[...]


def _matmul(h, wcat, b2d):
    return pl.pallas_call(
        _mm_body,
        grid=(N // BLK,),
        in_specs=[
            pl.BlockSpec((BLK, D), lambda i: (i, 0)),
            pl.BlockSpec((D, 2 * D), lambda i: (0, 0)),
            pl.BlockSpec((1, D), lambda i: (0, 0)),
        ],
        out_specs=[
            pl.BlockSpec((NC, BLK, HD), lambda i: (0, i, 0)),
            pl.BlockSpec((BLK, D), lambda i: (i, 0)),
        ],
        out_shape=[
            jax.ShapeDtypeStruct((NC, N, HD), jnp.float32),
            jax.ShapeDtypeStruct((N, D), jnp.float32),
        ],
    )(h, wcat, b2d)


def _ln_body(agg_ref, z_ref, g_ref, b_ref, h_ref, h2_ref):
    a = jnp.concatenate([agg_ref[0], agg_ref[1]], axis=1) + z_ref[...]
    m = jnp.mean(a, axis=1, keepdims=True)
    v = jnp.mean((a - m) ** 2, axis=1, keepdims=True)
    hn = (a - m) * lax.rsqrt(v + EPS) * g_ref[...] + b_ref[...]
    h_ref[...] = hn.astype(jnp.bfloat16)
    h2_ref[0] = hn[:, :HD]
    h2_ref[1] = hn[:, HD:]


def _layernorm(agg, z, g2d, b2d):
    return pl.pallas_call(
        _ln_body,
        grid=(N // BLK,),
        in_specs=[
            pl.BlockSpec((NC, BLK, HD), lambda i: (0, i, 0)),
            pl.BlockSpec((BLK, D), lambda i: (i, 0)),
            pl.BlockSpec((1, D), lambda i: (0, 0)),
            pl.BlockSpec((1, D), lambda i: (0, 0)),
        ],
        out_specs=[
            pl.BlockSpec((BLK, D), lambda i: (i, 0)),
            pl.BlockSpec((NC, BLK, HD), lambda i: (0, i, 0)),
        ],
        out_shape=[
            jax.ShapeDtypeStruct((N, D), jnp.bfloat16),
            jax.ShapeDtypeStruct((NC, N, HD), jnp.float32),
        ],
    )(agg, z, g2d, b2d)


_SC_MESH = plsc.VectorSubcoreMesh(core_axis_name="c", subcore_axis_name="s")


@functools.partial(
    pl.kernel,
    out_type=jax.ShapeDtypeStruct((NC, N, HD), jnp.float32),
    mesh=_SC_MESH,
    scratch_types=[
        pltpu.VMEM((IBLK, EK), jnp.int32),
        pltpu.VMEM((IBLK, EK), jnp.int32),
        pltpu.VMEM((2, EK, HD), jnp.float32),
        pltpu.VMEM_SHARED((N, HD), jnp.float32),
        pltpu.SemaphoreType.DMA((2,)),
    ],
)
def _edge_kernel(ytab, src4, dst4, zeros, agg_out, src_v, dst_v, rows_v, acc, sem):
    c = lax.axis_index("c")
    s = lax.axis_index("s")
    @pl.when(s < NS - 1)
    def _():
        pltpu.sync_copy(zeros.at[pl.ds(s * RPT, RPT)], acc.at[pl.ds(s * RPT, RPT)])

    @pl.when(s == NS - 1)
    def _():
        pltpu.sync_copy(zeros.at[pl.ds(s * RPT, RPT_LAST)],
                        acc.at[pl.ds(s * RPT, RPT_LAST)])

    off = c * N
    plsc.subcore_barrier()

    def _block(blk, _):
        pltpu.sync_copy(src4.at[s, blk], src_v)
        pltpu.sync_copy(dst4.at[s, blk], dst_v)

        def _add_off(i, _):
            for j in range(EK // 16):
                src_v[i, pl.ds(j * 16, 16)] = src_v[i, pl.ds(j * 16, 16)] + off
            return 0

        lax.fori_loop(0, IBLK, _add_off, 0)
        pltpu.async_copy(ytab.at[src_v.at[0]], rows_v.at[0], sem.at[0])

        def _chunk(i, _):
            b = lax.rem(i, 2)
            nb = 1 - b
            pltpu.make_async_copy(ytab.at[src_v.at[i]], rows_v.at[b], sem.at[b]).wait()

            @pl.when(i + 1 < IBLK)
            def _():
                pltpu.async_copy(ytab.at[src_v.at[i + 1]], rows_v.at[nb], sem.at[nb])

            pltpu.sync_copy(rows_v.at[b], acc.at[dst_v.at[i]], add=True)
            return 0

        lax.fori_loop(0, IBLK, _chunk, 0)
        return 0

    lax.fori_loop(0, NBLK, _block, 0)
    plsc.subcore_barrier()

    @pl.when(s < NS - 1)
    def _():
        pltpu.sync_copy(acc.at[pl.ds(s * RPT, RPT)],
                        agg_out.at[c, pl.ds(s * RPT, RPT)])

    @pl.when(s == NS - 1)
    def _():
        pltpu.sync_copy(acc.at[pl.ds(s * RPT, RPT_LAST)],
                        agg_out.at[c, pl.ds(s * RPT, RPT_LAST)])


@functools.partial(
    pl.kernel,
    out_type=(
        jax.ShapeDtypeStruct((NC, NG * HD), jnp.float32),
        jax.ShapeDtypeStruct((NC, NG * HD), jnp.float32),
    ),
    mesh=_SC_MESH,
    compiler_params=pltpu.CompilerParams(needs_layout_passes=False),
    scratch_types=[
        pltpu.VMEM((RPT // 2, HD), jnp.float32),
        pltpu.VMEM((RPT,), jnp.int32),
        pltpu.VMEM((NG * HD,), jnp.float32),
        pltpu.VMEM((NG * HD,), jnp.float32),
        pltpu.VMEM((NS, GPT * HD), jnp.float32),
        pltpu.VMEM((GPT * HD,), jnp.float32),
        pltpu.VMEM_SHARED((NS, NG * HD), jnp.float32),
    ],
)
def _pool_kernel(h2, batch, psum, pmax, rows_v, bid_v, sum_v, max_v, tmp_v, res_v, stage):
    c = lax.axis_index("c")
    s = lax.axis_index("s")
    base = s * RPT
    iota = lax.iota(jnp.int32, 16)
    z16 = jnp.zeros((16,), jnp.float32)
    neg16 = jnp.full((16,), NEG, jnp.float32)

    @pl.when(s < NS - 1)
    def _():
        pltpu.sync_copy(batch.at[pl.ds(base, RPT)], bid_v)

    @pl.when(s == NS - 1)
    def _():
        pltpu.sync_copy(batch.at[pl.ds(base, RPT_LAST)], bid_v.at[pl.ds(0, RPT_LAST)])

    def _init(i, _):
        sum_v[pl.ds(i * 16, 16)] = z16
        max_v[pl.ds(i * 16, 16)] = neg16
        return 0

    lax.fori_loop(0, NG * HD // 16, _init, 0)

    def _group_at(poff, gi):
        gbase = gi * 16
        bvec = bid_v[pl.ds(poff + gbase, 16)]
        same = jnp.max(bvec) == jnp.min(bvec)

        @pl.when(same)
        def _():
            rowbase = _splat_lane(bvec, 0) * HD
            for j in range(HD // 16):
                acc_s = rows_v[gbase, pl.ds(j * 16, 16)]
                acc_m = acc_s
                for r in range(1, 16):
                    v = rows_v[gbase + r, pl.ds(j * 16, 16)]
                    acc_s = acc_s + v
                    acc_m = jnp.maximum(acc_m, v)
                fi = rowbase + (j * 16 + iota)
                cs = plsc.load_gather(sum_v, [fi])
                plsc.store_scatter(sum_v, [fi], cs + acc_s)
                cm = plsc.load_gather(max_v, [fi])
                plsc.store_scatter(max_v, [fi], jnp.maximum(cm, acc_m))

        @pl.when(jnp.logical_not(same))
        def _():
            for r in range(16):
                rowbase = _splat_lane(bvec, r) * HD
                for j in range(HD // 16):
                    fi = rowbase + (j * 16 + iota)
                    v = rows_v[gbase + r, pl.ds(j * 16, 16)]
                    cs = plsc.load_gather(sum_v, [fi])
                    plsc.store_scatter(sum_v, [fi], cs + v)
                    cm = plsc.load_gather(max_v, [fi])
                    plsc.store_scatter(max_v, [fi], jnp.maximum(cm, v))

    RPB = RPT // 2

    pltpu.sync_copy(h2.at[c, pl.ds(base, RPB)], rows_v)

    def _g0(gi, _):
        _group_at(0, gi)
        return 0

    lax.fori_loop(0, RPB // 16, _g0, 0)

    @pl.when(s < NS - 1)
    def _():
        pltpu.sync_copy(h2.at[c, pl.ds(base + RPB, RPB)], rows_v)

    @pl.when(s == NS - 1)
    def _():
        pltpu.sync_copy(h2.at[c, pl.ds(base + RPB, RPT_LAST - RPB)],
                        rows_v.at[pl.ds(0, RPT_LAST - RPB)])

    ng1 = jnp.where(s < NS - 1, RPB // 16, (RPT_LAST - RPB) // 16)

    def _g1(gi, _):
        _group_at(RPB, gi)
        return 0

    lax.fori_loop(0, ng1, _g1, 0)

    seg = GPT * HD

    pltpu.sync_copy(sum_v, stage.at[s])
    plsc.subcore_barrier()

    @pl.when(s < NG // GPT)
    def _():
        pltpu.sync_copy(stage.at[pl.ds(0, NS), pl.ds(seg * s, seg)], tmp_v)

        def _zero(j, _):
            res_v[pl.ds(j * 16, 16)] = z16
            return 0

        lax.fori_loop(0, seg // 16, _zero, 0)

        def _msum(t, _):
            for j in range(seg // 16):
                res_v[pl.ds(j * 16, 16)] = (
                    res_v[pl.ds(j * 16, 16)] + tmp_v[t, pl.ds(j * 16, 16)])
            return 0

        lax.fori_loop(0, NS, _msum, 0)
        pltpu.sync_copy(res_v, psum.at[c, pl.ds(seg * s, seg)])

    plsc.subcore_barrier()
    pltpu.sync_copy(max_v, stage.at[s])
    plsc.subcore_barrier()

    @pl.when(s < NG // GPT)
    def _():
        pltpu.sync_copy(stage.at[pl.ds(0, NS), pl.ds(seg * s, seg)], tmp_v)

        def _zneg(j, _):
            res_v[pl.ds(j * 16, 16)] = neg16
            return 0

        lax.fori_loop(0, seg // 16, _zneg, 0)

        def _mmax(t, _):
            for j in range(seg // 16):
                res_v[pl.ds(j * 16, 16)] = jnp.maximum(
                    res_v[pl.ds(j * 16, 16)], tmp_v[t, pl.ds(j * 16, 16)])
            return 0

        lax.fori_loop(0, NS, _mmax, 0)
        pltpu.sync_copy(res_v, pmax.at[c, pl.ds(seg * s, seg)])


def _head_body(ps1, px1, ps2, px2, ps3, px3, bt, bng, bnb,
               w1, bl1, w2, bl2, w3, bl3, out_ref):
    gids = lax.broadcasted_iota(jnp.int32, (NG, 1), 0)
    cnt = jnp.sum((bt[...] == gids).astype(jnp.float32), axis=1, keepdims=True)
    cnt_c = jnp.maximum(cnt, 1.0)
    pieces = []
    for ps, px in ((ps1, px1), (ps2, px2), (ps3, px3)):
        sm = jnp.concatenate([ps[0], ps[1]], axis=1)
        mx = jnp.concatenate([px[0], px[1]], axis=1)
        pieces += [sm / cnt_c, jnp.where(cnt > 0.0, mx, 0.0), sm]
    hk = jnp.concatenate(pieces, axis=1)
    bm = jnp.mean(hk, axis=0, keepdims=True)
    bv = jnp.mean((hk - bm) ** 2, axis=0, keepdims=True)
    xn = (hk - bm) * lax.rsqrt(bv + EPS) * bng[...] + bnb[...]
    x1 = jnp.maximum(jnp.dot(xn, w1[...], preferred_element_type=jnp.float32) + bl1[...], 0.0)
    x2 = jnp.maximum(jnp.dot(x1, w2[...], preferred_element_type=jnp.float32) + bl2[...], 0.0)
    lg = jnp.dot(x2, w3[...], preferred_element_type=jnp.float32) + bl3[...]
    valid = lax.broadcasted_iota(jnp.int32, (NG, HD), 1) < NCLS
    mxl = jnp.max(jnp.where(valid, lg, NEG), axis=1, keepdims=True)
    ex = jnp.where(valid, jnp.exp(lg - mxl), 0.0)
    lse = jnp.log(jnp.sum(ex, axis=1, keepdims=True)) + mxl
    out_ref[...] = lg - lse


def _head(pools, batch2d, bng, bnb, w1, bl1, w2, bl2, w3p, bl3p):
    return pl.pallas_call(
        _head_body,
        out_shape=jax.ShapeDtypeStruct((NG, HD), jnp.float32),
    )(pools[0], pools[1], pools[2], pools[3], pools[4], pools[5],
      batch2d, bng, bnb, w1, bl1, w2, bl2, w3p, bl3p)


def kernel(x, edge_index, batch, W_rel1, W_root1, b1, W_rel2, W_root2, b2,
           ln_g, ln_b, bn_g, bn_b, W_l1, b_l1, W_l2, b_l2, W_l3, b_l3):
    src4 = edge_index[0].reshape(NS, NBLK, IBLK, EK)
    dst4 = edge_index[1].reshape(NS, NBLK, IBLK, EK)
    zeros_nh = jnp.zeros((N, HD), jnp.float32)
    wcat1 = jnp.concatenate([W_rel1, W_root1], axis=1)
    wcat2 = jnp.concatenate([W_rel2, W_root2], axis=1)
    b1_2d = b1.reshape(1, D)
    b2_2d = b2.reshape(1, D)
    lng = ln_g.reshape(1, D)
    lnb = ln_b.reshape(1, D)

    h = x.astype(jnp.bfloat16)
    wcat1 = wcat1.astype(jnp.bfloat16)
    wcat2 = wcat2.astype(jnp.bfloat16)
    pools = []
    for k in range(3):
        wcat, bb = (wcat1, b1_2d) if k == 0 else (wcat2, b2_2d)
        yrel, z = _matmul(h, wcat, bb)
        agg = _edge_kernel(yrel.reshape(NC * N, HD), src4, dst4, zeros_nh)
        h, h2 = _layernorm(agg, z, lng, lnb)
        ps, px = _pool_kernel(h2, batch)
        pools += [ps.reshape(NC, NG, HD), px.reshape(NC, NG, HD)]

    w3p = jnp.pad(W_l3, ((0, 0), (0, HD - NCLS)))
    bl3p = jnp.pad(b_l3, (0, HD - NCLS)).reshape(1, HD)
    out128 = _head(pools, batch.reshape(1, N), bn_g.reshape(1, -1),
                   bn_b.reshape(1, -1), W_l1, b_l1.reshape(1, -1),
                   W_l2, b_l2.reshape(1, -1), w3p, bl3p)
    return out128[:, :NCLS]


# speedup vs baseline: 6.1706x; 1.2664x over previous
import functools

import numpy as np

import jax
import jax.numpy as jnp
from jax import lax
from jax.experimental import pallas as pl
from jax.experimental.pallas import tpu as pltpu
from jax.experimental.pallas import tpu_sc as plsc

N = 10000
E = 160000
D = 256
HD = 128
NG = 64
NCLS = 2
NC, NS = 2, 16
EPS = 1e-5
NEG = -3.4e38

EK = 80
EPT = E // NS
NCHUNK = EPT // EK
IBLK = 25
NBLK = NCHUNK // IBLK
RPT = 640
RPT_LAST = N - (NS - 1) * RPT
GPT = 8

BLK = 1000

_GDN = lax.GatherDimensionNumbers(
    offset_dims=(), collapsed_slice_dims=(0,), start_index_map=(0,))


def _splat_lane(vec, r):
    idx = jnp.full((16, 1), r, jnp.int32)
    return lax.gather(vec, idx, dimension_numbers=_GDN, slice_sizes=(1,),
                      mode=lax.GatherScatterMode.PROMISE_IN_BOUNDS)


def _mm_body(h_ref, w_ref, b_ref, yrel_ref, z_ref):
    y = jnp.dot(h_ref[...], w_ref[...], preferred_element_type=jnp.float32)
    yrel_ref[0] = y[:, :HD]
    yrel_ref[1] = y[:, HD:D]
    z_ref[...] = y[:, D:] + b_ref[...]


def _matmul(h, wcat, b2d):
    return pl.pallas_call(
        _mm_body,
        grid=(N // BLK,),
        in_specs=[
            pl.BlockSpec((BLK, D), lambda i: (i, 0)),
            pl.BlockSpec((D, 2 * D), lambda i: (0, 0)),
            pl.BlockSpec((1, D), lambda i: (0, 0)),
        ],
        out_specs=[
            pl.BlockSpec((NC, BLK, HD), lambda i: (0, i, 0)),
            pl.BlockSpec((BLK, D), lambda i: (i, 0)),
        ],
        out_shape=[
            jax.ShapeDtypeStruct((NC, N, HD), jnp.float32),
            jax.ShapeDtypeStruct((N, D), jnp.float32),
        ],
    )(h, wcat, b2d)


def _ln_body(agg_ref, z_ref, g_ref, b_ref, h_ref, h2_ref):
    a = jnp.concatenate([agg_ref[0], agg_ref[1]], axis=1) + z_ref[...]
    m = jnp.mean(a, axis=1, keepdims=True)
    v = jnp.mean((a - m) ** 2, axis=1, keepdims=True)
    hn = (a - m) * lax.rsqrt(v + EPS) * g_ref[...] + b_ref[...]
    h_ref[...] = hn.astype(jnp.bfloat16)
    h2_ref[0] = hn[:, :HD]
    h2_ref[1] = hn[:, HD:]


def _layernorm(agg, z, g2d, b2d):
    return pl.pallas_call(
        _ln_body,
        grid=(N // BLK,),
        in_specs=[
            pl.BlockSpec((NC, BLK, HD), lambda i: (0, i, 0)),
            pl.BlockSpec((BLK, D), lambda i: (i, 0)),
            pl.BlockSpec((1, D), lambda i: (0, 0)),
            pl.BlockSpec((1, D), lambda i: (0, 0)),
        ],
        out_specs=[
            pl.BlockSpec((BLK, D), lambda i: (i, 0)),
            pl.BlockSpec((NC, BLK, HD), lambda i: (0, i, 0)),
        ],
        out_shape=[
            jax.ShapeDtypeStruct((N, D), jnp.bfloat16),
            jax.ShapeDtypeStruct((NC, N, HD), jnp.float32),
        ],
    )(agg, z, g2d, b2d)


_SC_MESH = plsc.VectorSubcoreMesh(core_axis_name="c", subcore_axis_name="s")


@functools.partial(
    pl.kernel,
    out_type=jax.ShapeDtypeStruct((NC, N, HD), jnp.float32),
    mesh=_SC_MESH,
    compiler_params=pltpu.CompilerParams(needs_layout_passes=False),
    scratch_types=[
        pltpu.VMEM((IBLK, EK), jnp.int32),
        pltpu.VMEM((IBLK, EK), jnp.int32),
        pltpu.VMEM((3, EK, HD), jnp.float32),
        pltpu.VMEM_SHARED((N, HD), jnp.float32),
        pltpu.SemaphoreType.DMA((3,)),
        pltpu.SemaphoreType.DMA((3,)),
    ],
)
def _edge_kernel(ytab, src4, dst4, zeros, agg_out, src_v, dst_v, rows_v, acc, sem_g, sem_s):
    c = lax.axis_index("c")
    s = lax.axis_index("s")
    @pl.when(s < NS - 1)
    def _():
        pltpu.sync_copy(zeros.at[pl.ds(s * RPT, RPT)], acc.at[pl.ds(s * RPT, RPT)])

    @pl.when(s == NS - 1)
    def _():
        pltpu.sync_copy(zeros.at[pl.ds(s * RPT, RPT_LAST)],
                        acc.at[pl.ds(s * RPT, RPT_LAST)])

    off = c * N
    plsc.subcore_barrier()

    def _block(blk, _):
        pltpu.sync_copy(src4.at[s, blk], src_v)
        pltpu.sync_copy(dst4.at[s, blk], dst_v)

        def _add_off(i, _):
            for j in range(EK // 16):
                src_v[i, pl.ds(j * 16, 16)] = src_v[i, pl.ds(j * 16, 16)] + off
            return 0

        lax.fori_loop(0, IBLK, _add_off, 0)

        pltpu.async_copy(ytab.at[src_v.at[0]], rows_v.at[0], sem_g.at[0])
        pltpu.async_copy(ytab.at[src_v.at[1]], rows_v.at[1], sem_g.at[1])

        def _chunk(i, _):
            sl = lax.rem(i, 3)
            pltpu.make_async_copy(ytab.at[src_v.at[i]], rows_v.at[sl],
                                  sem_g.at[sl]).wait()
            pltpu.async_copy(rows_v.at[sl], acc.at[dst_v.at[i]],
                             sem_s.at[sl], add=True)

            @pl.when(i + 2 < IBLK)
            def _():
                sl2 = lax.rem(i + 2, 3)

                @pl.when(i >= 1)
                def _():
                    pltpu.make_async_copy(rows_v.at[sl2], acc.at[dst_v.at[0]],
                                          sem_s.at[sl2]).wait()

                pltpu.async_copy(ytab.at[src_v.at[i + 2]], rows_v.at[sl2],
                                 sem_g.at[sl2])
            return 0

        lax.fori_loop(0, IBLK, _chunk, 0)
        for k in range(3):
            pltpu.make_async_copy(rows_v.at[k], acc.at[dst_v.at[0]],
                                  sem_s.at[k]).wait()
        return 0

    lax.fori_loop(0, NBLK, _block, 0)
    plsc.subcore_barrier()

    @pl.when(s < NS - 1)
    def _():
        pltpu.sync_copy(acc.at[pl.ds(s * RPT, RPT)],
                        agg_out.at[c, pl.ds(s * RPT, RPT)])

    @pl.when(s == NS - 1)
    def _():
        pltpu.sync_copy(acc.at[pl.ds(s * RPT, RPT_LAST)],
                        agg_out.at[c, pl.ds(s * RPT, RPT_LAST)])


@functools.partial(
    pl.kernel,
    out_type=(
        jax.ShapeDtypeStruct((NC, NG * HD), jnp.float32),
        jax.ShapeDtypeStruct((NC, NG * HD), jnp.float32),
    ),
    mesh=_SC_MESH,
    compiler_params=pltpu.CompilerParams(needs_layout_passes=False),
    scratch_types=[
        pltpu.VMEM((RPT // 2, HD), jnp.float32),
        pltpu.VMEM((RPT,), jnp.int32),
        pltpu.VMEM((NG * HD,), jnp.float32),
        pltpu.VMEM((NG * HD,), jnp.float32),
        pltpu.VMEM((NS, GPT * HD), jnp.float32),
        pltpu.VMEM((GPT * HD,), jnp.float32),
        pltpu.VMEM_SHARED((NS, NG * HD), jnp.float32),
    ],
)
def _pool_kernel(h2, batch, psum, pmax, rows_v, bid_v, sum_v, max_v, tmp_v, res_v, stage):
    c = lax.axis_index("c")
    s = lax.axis_index("s")
    base = s * RPT
    iota = lax.iota(jnp.int32, 16)
    z16 = jnp.zeros((16,), jnp.float32)
    neg16 = jnp.full((16,), NEG, jnp.float32)

    @pl.when(s < NS - 1)
    def _():
        pltpu.sync_copy(batch.at[pl.ds(base, RPT)], bid_v)

    @pl.when(s == NS - 1)
    def _():
        pltpu.sync_copy(batch.at[pl.ds(base, RPT_LAST)], bid_v.at[pl.ds(0, RPT_LAST)])

    def _init(i, _):
        sum_v[pl.ds(i * 16, 16)] = z16
        max_v[pl.ds(i * 16, 16)] = neg16
        return 0

    lax.fori_loop(0, NG * HD // 16, _init, 0)

    def _group_at(poff, gi):
        gbase = gi * 16
        bvec = bid_v[pl.ds(poff + gbase, 16)]
        same = jnp.max(bvec) == jnp.min(bvec)

        @pl.when(same)
        def _():
            rowbase = _splat_lane(bvec, 0) * HD
            for j in range(HD // 16):
                acc_s = rows_v[gbase, pl.ds(j * 16, 16)]
                acc_m = acc_s
                for r in range(1, 16):
                    v = rows_v[gbase + r, pl.ds(j * 16, 16)]
                    acc_s = acc_s + v
                    acc_m = jnp.maximum(acc_m, v)
                fi = rowbase + (j * 16 + iota)
                cs = plsc.load_gather(sum_v, [fi])
                plsc.store_scatter(sum_v, [fi], cs + acc_s)
                cm = plsc.load_gather(max_v, [fi])
                plsc.store_scatter(max_v, [fi], jnp.maximum(cm, acc_m))

        @pl.when(jnp.logical_not(same))
        def _():
            for r in range(16):
                rowbase = _splat_lane(bvec, r) * HD
                for j in range(HD // 16):
                    fi = rowbase + (j * 16 + iota)
                    v = rows_v[gbase + r, pl.ds(j * 16, 16)]
                    cs = plsc.load_gather(sum_v, [fi])
                    plsc.store_scatter(sum_v, [fi], cs + v)
                    cm = plsc.load_gather(max_v, [fi])
                    plsc.store_scatter(max_v, [fi], jnp.maximum(cm, v))

    RPB = RPT // 2

    pltpu.sync_copy(h2.at[c, pl.ds(base, RPB)], rows_v)

    def _g0(gi, _):
        _group_at(0, gi)
        return 0

    lax.fori_loop(0, RPB // 16, _g0, 0)

    @pl.when(s < NS - 1)
    def _():
        pltpu.sync_copy(h2.at[c, pl.ds(base + RPB, RPB)], rows_v)

    @pl.when(s == NS - 1)
    def _():
        pltpu.sync_copy(h2.at[c, pl.ds(base + RPB, RPT_LAST - RPB)],
                        rows_v.at[pl.ds(0, RPT_LAST - RPB)])

    ng1 = jnp.where(s < NS - 1, RPB // 16, (RPT_LAST - RPB) // 16)

    def _g1(gi, _):
        _group_at(RPB, gi)
        return 0

    lax.fori_loop(0, ng1, _g1, 0)

    seg = GPT * HD

    pltpu.sync_copy(sum_v, stage.at[s])
    plsc.subcore_barrier()

    @pl.when(s < NG // GPT)
    def _():
        pltpu.sync_copy(stage.at[pl.ds(0, NS), pl.ds(seg * s, seg)], tmp_v)

        def _zero(j, _):
            res_v[pl.ds(j * 16, 16)] = z16
            return 0

        lax.fori_loop(0, seg // 16, _zero, 0)

        def _msum(t, _):
            for j in range(seg // 16):
                res_v[pl.ds(j * 16, 16)] = (
                    res_v[pl.ds(j * 16, 16)] + tmp_v[t, pl.ds(j * 16, 16)])
            return 0

        lax.fori_loop(0, NS, _msum, 0)
        pltpu.sync_copy(res_v, psum.at[c, pl.ds(seg * s, seg)])

    plsc.subcore_barrier()
    pltpu.sync_copy(max_v, stage.at[s])
    plsc.subcore_barrier()

    @pl.when(s < NG // GPT)
    def _():
        pltpu.sync_copy(stage.at[pl.ds(0, NS), pl.ds(seg * s, seg)], tmp_v)

        def _zneg(j, _):
            res_v[pl.ds(j * 16, 16)] = neg16
            return 0

        lax.fori_loop(0, seg // 16, _zneg, 0)

        def _mmax(t, _):
            for j in range(seg // 16):
                res_v[pl.ds(j * 16, 16)] = jnp.maximum(
                    res_v[pl.ds(j * 16, 16)], tmp_v[t, pl.ds(j * 16, 16)])
            return 0

        lax.fori_loop(0, NS, _mmax, 0)
        pltpu.sync_copy(res_v, pmax.at[c, pl.ds(seg * s, seg)])


def _head_body(ps1, px1, ps2, px2, ps3, px3, bt, bng, bnb,
               w1, bl1, w2, bl2, w3, bl3, out_ref):
    gids = lax.broadcasted_iota(jnp.int32, (NG, 1), 0)
    cnt = jnp.sum((bt[...] == gids).astype(jnp.float32), axis=1, keepdims=True)
    cnt_c = jnp.maximum(cnt, 1.0)
    pieces = []
    for ps, px in ((ps1, px1), (ps2, px2), (ps3, px3)):
        sm = jnp.concatenate([ps[0], ps[1]], axis=1)
        mx = jnp.concatenate([px[0], px[1]], axis=1)
        pieces += [sm / cnt_c, jnp.where(cnt > 0.0, mx, 0.0), sm]
    hk = jnp.concatenate(pieces, axis=1)
    bm = jnp.mean(hk, axis=0, keepdims=True)
    bv = jnp.mean((hk - bm) ** 2, axis=0, keepdims=True)
    xn = (hk - bm) * lax.rsqrt(bv + EPS) * bng[...] + bnb[...]
    x1 = jnp.maximum(jnp.dot(xn, w1[...], preferred_element_type=jnp.float32) + bl1[...], 0.0)
    x2 = jnp.maximum(jnp.dot(x1, w2[...], preferred_element_type=jnp.float32) + bl2[...], 0.0)
    lg = jnp.dot(x2, w3[...], preferred_element_type=jnp.float32) + bl3[...]
    valid = lax.broadcasted_iota(jnp.int32, (NG, HD), 1) < NCLS
    mxl = jnp.max(jnp.where(valid, lg, NEG), axis=1, keepdims=True)
    ex = jnp.where(valid, jnp.exp(lg - mxl), 0.0)
    lse = jnp.log(jnp.sum(ex, axis=1, keepdims=True)) + mxl
    out_ref[...] = lg - lse


def _head(pools, batch2d, bng, bnb, w1, bl1, w2, bl2, w3p, bl3p):
    return pl.pallas_call(
        _head_body,
        out_shape=jax.ShapeDtypeStruct((NG, HD), jnp.float32),
    )(pools[0], pools[1], pools[2], pools[3], pools[4], pools[5],
      batch2d, bng, bnb, w1, bl1, w2, bl2, w3p, bl3p)


def kernel(x, edge_index, batch, W_rel1, W_root1, b1, W_rel2, W_root2, b2,
           ln_g, ln_b, bn_g, bn_b, W_l1, b_l1, W_l2, b_l2, W_l3, b_l3):
    src4 = edge_index[0].reshape(NS, NBLK, IBLK, EK)
    dst4 = edge_index[1].reshape(NS, NBLK, IBLK, EK)
    zeros_nh = jnp.zeros((N, HD), jnp.float32)
    wcat1 = jnp.concatenate([W_rel1, W_root1], axis=1)
    wcat2 = jnp.concatenate([W_rel2, W_root2], axis=1)
    b1_2d = b1.reshape(1, D)
    b2_2d = b2.reshape(1, D)
    lng = ln_g.reshape(1, D)
    lnb = ln_b.reshape(1, D)

    h = x.astype(jnp.bfloat16)
    wcat1 = wcat1.astype(jnp.bfloat16)
    wcat2 = wcat2.astype(jnp.bfloat16)
    pools = []
    for k in range(3):
        wcat, bb = (wcat1, b1_2d) if k == 0 else (wcat2, b2_2d)
        yrel, z = _matmul(h, wcat, bb)
        agg = _edge_kernel(yrel.reshape(NC * N, HD), src4, dst4, zeros_nh)
        h, h2 = _layernorm(agg, z, lng, lnb)
        ps, px = _pool_kernel(h2, batch)
        pools += [ps.reshape(NC, NG, HD), px.reshape(NC, NG, HD)]

    w3p = jnp.pad(W_l3, ((0, 0), (0, HD - NCLS)))
    bl3p = jnp.pad(b_l3, (0, HD - NCLS)).reshape(1, HD)
    out128 = _head(pools, batch.reshape(1, N), bn_g.reshape(1, -1),
                   bn_b.reshape(1, -1), W_l1, b_l1.reshape(1, -1),
                   W_l2, b_l2.reshape(1, -1), w3p, bl3p)
    return out128[:, :NCLS]


# speedup vs baseline: 6.3526x; 1.0295x over previous
import functools

import numpy as np

import jax
import jax.numpy as jnp
from jax import lax
from jax.experimental import pallas as pl
from jax.experimental.pallas import tpu as pltpu
from jax.experimental.pallas import tpu_sc as plsc

N = 10000
E = 160000
D = 256
HD = 128
NG = 64
NCLS = 2
NC, NS = 2, 16
EPS = 1e-5
NEG = -3.4e38

EK = 80
EPT = E // NS
NCHUNK = EPT // EK
IBLK = 25
NBLK = NCHUNK // IBLK
RPT = 640
RPT_LAST = N - (NS - 1) * RPT
GPT = 8

BLK = 1000

_GDN = lax.GatherDimensionNumbers(
    offset_dims=(), collapsed_slice_dims=(0,), start_index_map=(0,))


def _splat_lane(vec, r):
    idx = jnp.full((16, 1), r, jnp.int32)
    return lax.gather(vec, idx, dimension_numbers=_GDN, slice_sizes=(1,),
                      mode=lax.GatherScatterMode.PROMISE_IN_BOUNDS)


def _mm_body(h_ref, w_ref, b_ref, yrel_ref, z_ref):
    y = jnp.dot(h_ref[...], w_ref[...], preferred_element_type=jnp.float32)
    yrel_ref[0] = y[:, :HD]
    yrel_ref[1] = y[:, HD:D]
    z_ref[...] = y[:, D:] + b_ref[...]


def _matmul(h, wcat, b2d):
    return pl.pallas_call(
        _mm_body,
        grid=(N // BLK,),
        in_specs=[
            pl.BlockSpec((BLK, D), lambda i: (i, 0)),
            pl.BlockSpec((D, 2 * D), lambda i: (0, 0)),
            pl.BlockSpec((1, D), lambda i: (0, 0)),
        ],
        out_specs=[
            pl.BlockSpec((NC, BLK, HD), lambda i: (0, i, 0)),
            pl.BlockSpec((BLK, D), lambda i: (i, 0)),
        ],
        out_shape=[
            jax.ShapeDtypeStruct((NC, N, HD), jnp.float32),
            jax.ShapeDtypeStruct((N, D), jnp.float32),
        ],
    )(h, wcat, b2d)


def _ln_body(agg_ref, z_ref, g_ref, b_ref, h_ref, h2_ref):
    a = jnp.concatenate([agg_ref[0], agg_ref[1]], axis=1) + z_ref[...]
    m = jnp.mean(a, axis=1, keepdims=True)
    v = jnp.mean((a - m) ** 2, axis=1, keepdims=True)
    hn = (a - m) * lax.rsqrt(v + EPS) * g_ref[...] + b_ref[...]
    h_ref[...] = hn.astype(jnp.bfloat16)
    h2_ref[0] = hn[:, :HD]
    h2_ref[1] = hn[:, HD:]


def _layernorm(agg, z, g2d, b2d):
    return pl.pallas_call(
        _ln_body,
        grid=(N // BLK,),
        in_specs=[
            pl.BlockSpec((NC, BLK, HD), lambda i: (0, i, 0)),
            pl.BlockSpec((BLK, D), lambda i: (i, 0)),
            pl.BlockSpec((1, D), lambda i: (0, 0)),
            pl.BlockSpec((1, D), lambda i: (0, 0)),
        ],
        out_specs=[
            pl.BlockSpec((BLK, D), lambda i: (i, 0)),
            pl.BlockSpec((NC, BLK, HD), lambda i: (0, i, 0)),
        ],
        out_shape=[
            jax.ShapeDtypeStruct((N, D), jnp.bfloat16),
            jax.ShapeDtypeStruct((NC, N, HD), jnp.float32),
        ],
    )(agg, z, g2d, b2d)


def _lnmm_body(agg_ref, z_ref, g_ref, b_ref, w_ref, bn_ref,
               h2_ref, yrel_ref, zn_ref):
    a = jnp.concatenate([agg_ref[0], agg_ref[1]], axis=1) + z_ref[...]
    m = jnp.mean(a, axis=1, keepdims=True)
    v = jnp.mean((a - m) ** 2, axis=1, keepdims=True)
    hn = (a - m) * lax.rsqrt(v + EPS) * g_ref[...] + b_ref[...]
    h2_ref[0] = hn[:, :HD]
    h2_ref[1] = hn[:, HD:]
    y = jnp.dot(hn.astype(jnp.bfloat16), w_ref[...],
                preferred_element_type=jnp.float32)
    yrel_ref[0] = y[:, :HD]
    yrel_ref[1] = y[:, HD:D]
    zn_ref[...] = y[:, D:] + bn_ref[...]


def _ln_matmul(agg, z, g2d, b2d, wcat, bn2d):
    return pl.pallas_call(
        _lnmm_body,
        grid=(N // BLK,),
        in_specs=[
            pl.BlockSpec((NC, BLK, HD), lambda i: (0, i, 0)),
            pl.BlockSpec((BLK, D), lambda i: (i, 0)),
            pl.BlockSpec((1, D), lambda i: (0, 0)),
            pl.BlockSpec((1, D), lambda i: (0, 0)),
            pl.BlockSpec((D, 2 * D), lambda i: (0, 0)),
            pl.BlockSpec((1, D), lambda i: (0, 0)),
        ],
        out_specs=[
            pl.BlockSpec((NC, BLK, HD), lambda i: (0, i, 0)),
            pl.BlockSpec((NC, BLK, HD), lambda i: (0, i, 0)),
            pl.BlockSpec((BLK, D), lambda i: (i, 0)),
        ],
        out_shape=[
            jax.ShapeDtypeStruct((NC, N, HD), jnp.float32),
            jax.ShapeDtypeStruct((NC, N, HD), jnp.float32),
            jax.ShapeDtypeStruct((N, D), jnp.float32),
        ],
    )(agg, z, g2d, b2d, wcat, bn2d)


_SC_MESH = plsc.VectorSubcoreMesh(core_axis_name="c", subcore_axis_name="s")


@functools.partial(
    pl.kernel,
    out_type=jax.ShapeDtypeStruct((NC, N, HD), jnp.float32),
    mesh=_SC_MESH,
    compiler_params=pltpu.CompilerParams(needs_layout_passes=False),
    scratch_types=[
        pltpu.VMEM((IBLK, EK), jnp.int32),
        pltpu.VMEM((IBLK, EK), jnp.int32),
        pltpu.VMEM((3, EK, HD), jnp.float32),
        pltpu.VMEM_SHARED((N, HD), jnp.float32),
        pltpu.SemaphoreType.DMA((3,)),
        pltpu.SemaphoreType.DMA((3,)),
    ],
)
def _edge_kernel(ytab, src4, dst4, zeros, agg_out, src_v, dst_v, rows_v, acc, sem_g, sem_s):
    c = lax.axis_index("c")
    s = lax.axis_index("s")
    @pl.when(s < NS - 1)
    def _():
        pltpu.sync_copy(zeros.at[pl.ds(s * RPT, RPT)], acc.at[pl.ds(s * RPT, RPT)])

    @pl.when(s == NS - 1)
    def _():
        pltpu.sync_copy(zeros.at[pl.ds(s * RPT, RPT_LAST)],
                        acc.at[pl.ds(s * RPT, RPT_LAST)])

    off = c * N
    plsc.subcore_barrier()

    def _block(blk, _):
        pltpu.sync_copy(src4.at[s, blk], src_v)
        pltpu.sync_copy(dst4.at[s, blk], dst_v)

        def _add_off(i, _):
            for j in range(EK // 16):
                src_v[i, pl.ds(j * 16, 16)] = src_v[i, pl.ds(j * 16, 16)] + off
            return 0

        lax.fori_loop(0, IBLK, _add_off, 0)

        pltpu.async_copy(ytab.at[src_v.at[0]], rows_v.at[0], sem_g.at[0])
        pltpu.async_copy(ytab.at[src_v.at[1]], rows_v.at[1], sem_g.at[1])

        def _chunk(i, _):
            sl = lax.rem(i, 3)
            pltpu.make_async_copy(ytab.at[src_v.at[i]], rows_v.at[sl],
                                  sem_g.at[sl]).wait()
            pltpu.async_copy(rows_v.at[sl], acc.at[dst_v.at[i]],
                             sem_s.at[sl], add=True)

            @pl.when(i + 2 < IBLK)
            def _():
                sl2 = lax.rem(i + 2, 3)

                @pl.when(i >= 1)
                def _():
                    pltpu.make_async_copy(rows_v.at[sl2], acc.at[dst_v.at[0]],
                                          sem_s.at[sl2]).wait()

                pltpu.async_copy(ytab.at[src_v.at[i + 2]], rows_v.at[sl2],
                                 sem_g.at[sl2])
            return 0

        lax.fori_loop(0, IBLK, _chunk, 0)
        for k in range(3):
            pltpu.make_async_copy(rows_v.at[k], acc.at[dst_v.at[0]],
                                  sem_s.at[k]).wait()
        return 0

    lax.fori_loop(0, NBLK, _block, 0)
    plsc.subcore_barrier()

    @pl.when(s < NS - 1)
    def _():
        pltpu.sync_copy(acc.at[pl.ds(s * RPT, RPT)],
                        agg_out.at[c, pl.ds(s * RPT, RPT)])

    @pl.when(s == NS - 1)
    def _():
        pltpu.sync_copy(acc.at[pl.ds(s * RPT, RPT_LAST)],
                        agg_out.at[c, pl.ds(s * RPT, RPT_LAST)])


@functools.partial(
    pl.kernel,
    out_type=(
        jax.ShapeDtypeStruct((NC, NG * HD), jnp.float32),
        jax.ShapeDtypeStruct((NC, NG * HD), jnp.float32),
    ),
    mesh=_SC_MESH,
    compiler_params=pltpu.CompilerParams(needs_layout_passes=False),
    scratch_types=[
        pltpu.VMEM((RPT // 2, HD), jnp.float32),
        pltpu.VMEM((RPT,), jnp.int32),
        pltpu.VMEM((NG * HD,), jnp.float32),
        pltpu.VMEM((NG * HD,), jnp.float32),
        pltpu.VMEM((NS, GPT * HD), jnp.float32),
        pltpu.VMEM((GPT * HD,), jnp.float32),
        pltpu.VMEM_SHARED((NS, NG * HD), jnp.float32),
    ],
)
def _pool_kernel(h2, batch, psum, pmax, rows_v, bid_v, sum_v, max_v, tmp_v, res_v, stage):
    c = lax.axis_index("c")
    s = lax.axis_index("s")
    base = s * RPT
    iota = lax.iota(jnp.int32, 16)
    z16 = jnp.zeros((16,), jnp.float32)
    neg16 = jnp.full((16,), NEG, jnp.float32)

    @pl.when(s < NS - 1)
    def _():
        pltpu.sync_copy(batch.at[pl.ds(base, RPT)], bid_v)

    @pl.when(s == NS - 1)
    def _():
        pltpu.sync_copy(batch.at[pl.ds(base, RPT_LAST)], bid_v.at[pl.ds(0, RPT_LAST)])

    def _init(i, _):
        sum_v[pl.ds(i * 16, 16)] = z16
        max_v[pl.ds(i * 16, 16)] = neg16
        return 0

    lax.fori_loop(0, NG * HD // 16, _init, 0)

    def _group_at(poff, gi):
        gbase = gi * 16
        bvec = bid_v[pl.ds(poff + gbase, 16)]
        same = jnp.max(bvec) == jnp.min(bvec)

        @pl.when(same)
        def _():
            rowbase = _splat_lane(bvec, 0) * HD
            for j in range(HD // 16):
                acc_s = rows_v[gbase, pl.ds(j * 16, 16)]
                acc_m = acc_s
                for r in range(1, 16):
                    v = rows_v[gbase + r, pl.ds(j * 16, 16)]
                    acc_s = acc_s + v
                    acc_m = jnp.maximum(acc_m, v)
                fi = rowbase + (j * 16 + iota)
                cs = plsc.load_gather(sum_v, [fi])
                plsc.store_scatter(sum_v, [fi], cs + acc_s)
                cm = plsc.load_gather(max_v, [fi])
                plsc.store_scatter(max_v, [fi], jnp.maximum(cm, acc_m))

        @pl.when(jnp.logical_not(same))
        def _():
            for r in range(16):
                rowbase = _splat_lane(bvec, r) * HD
                for j in range(HD // 16):
                    fi = rowbase + (j * 16 + iota)
                    v = rows_v[gbase + r, pl.ds(j * 16, 16)]
                    cs = plsc.load_gather(sum_v, [fi])
                    plsc.store_scatter(sum_v, [fi], cs + v)
                    cm = plsc.load_gather(max_v, [fi])
                    plsc.store_scatter(max_v, [fi], jnp.maximum(cm, v))

    RPB = RPT // 2

    pltpu.sync_copy(h2.at[c, pl.ds(base, RPB)], rows_v)

    def _g0(gi, _):
        _group_at(0, gi)
        return 0

    lax.fori_loop(0, RPB // 16, _g0, 0)

    @pl.when(s < NS - 1)
    def _():
        pltpu.sync_copy(h2.at[c, pl.ds(base + RPB, RPB)], rows_v)

    @pl.when(s == NS - 1)
    def _():
        pltpu.sync_copy(h2.at[c, pl.ds(base + RPB, RPT_LAST - RPB)],
                        rows_v.at[pl.ds(0, RPT_LAST - RPB)])

    ng1 = jnp.where(s < NS - 1, RPB // 16, (RPT_LAST - RPB) // 16)

    def _g1(gi, _):
        _group_at(RPB, gi)
        return 0

    lax.fori_loop(0, ng1, _g1, 0)

    seg = GPT * HD

    pltpu.sync_copy(sum_v, stage.at[s])
    plsc.subcore_barrier()

    @pl.when(s < NG // GPT)
    def _():
        pltpu.sync_copy(stage.at[pl.ds(0, NS), pl.ds(seg * s, seg)], tmp_v)

        def _zero(j, _):
            res_v[pl.ds(j * 16, 16)] = z16
            return 0

        lax.fori_loop(0, seg // 16, _zero, 0)

        def _msum(t, _):
            for j in range(seg // 16):
                res_v[pl.ds(j * 16, 16)] = (
                    res_v[pl.ds(j * 16, 16)] + tmp_v[t, pl.ds(j * 16, 16)])
            return 0

        lax.fori_loop(0, NS, _msum, 0)
        pltpu.sync_copy(res_v, psum.at[c, pl.ds(seg * s, seg)])

    plsc.subcore_barrier()
    pltpu.sync_copy(max_v, stage.at[s])
    plsc.subcore_barrier()

    @pl.when(s < NG // GPT)
    def _():
        pltpu.sync_copy(stage.at[pl.ds(0, NS), pl.ds(seg * s, seg)], tmp_v)

        def _zneg(j, _):
            res_v[pl.ds(j * 16, 16)] = neg16
            return 0

        lax.fori_loop(0, seg // 16, _zneg, 0)

        def _mmax(t, _):
            for j in range(seg // 16):
                res_v[pl.ds(j * 16, 16)] = jnp.maximum(
                    res_v[pl.ds(j * 16, 16)], tmp_v[t, pl.ds(j * 16, 16)])
            return 0

        lax.fori_loop(0, NS, _mmax, 0)
        pltpu.sync_copy(res_v, pmax.at[c, pl.ds(seg * s, seg)])


def _head_body(ps1, px1, ps2, px2, ps3, px3, bt, bng, bnb,
               w1, bl1, w2, bl2, w3, bl3, out_ref):
    gids = lax.broadcasted_iota(jnp.int32, (NG, 1), 0)
    cnt = jnp.sum((bt[...] == gids).astype(jnp.float32), axis=1, keepdims=True)
    cnt_c = jnp.maximum(cnt, 1.0)
    pieces = []
    for ps, px in ((ps1, px1), (ps2, px2), (ps3, px3)):
        sm = jnp.concatenate([ps[0], ps[1]], axis=1)
        mx = jnp.concatenate([px[0], px[1]], axis=1)
        pieces += [sm / cnt_c, jnp.where(cnt > 0.0, mx, 0.0), sm]
    hk = jnp.concatenate(pieces, axis=1)
    bm = jnp.mean(hk, axis=0, keepdims=True)
    bv = jnp.mean((hk - bm) ** 2, axis=0, keepdims=True)
    xn = (hk - bm) * lax.rsqrt(bv + EPS) * bng[...] + bnb[...]
    x1 = jnp.maximum(jnp.dot(xn, w1[...], preferred_element_type=jnp.float32) + bl1[...], 0.0)
    x2 = jnp.maximum(jnp.dot(x1, w2[...], preferred_element_type=jnp.float32) + bl2[...], 0.0)
    lg = jnp.dot(x2, w3[...], preferred_element_type=jnp.float32) + bl3[...]
    valid = lax.broadcasted_iota(jnp.int32, (NG, HD), 1) < NCLS
    mxl = jnp.max(jnp.where(valid, lg, NEG), axis=1, keepdims=True)
    ex = jnp.where(valid, jnp.exp(lg - mxl), 0.0)
    lse = jnp.log(jnp.sum(ex, axis=1, keepdims=True)) + mxl
    out_ref[...] = lg - lse


def _head(pools, batch2d, bng, bnb, w1, bl1, w2, bl2, w3p, bl3p):
    return pl.pallas_call(
        _head_body,
        out_shape=jax.ShapeDtypeStruct((NG, HD), jnp.float32),
    )(pools[0], pools[1], pools[2], pools[3], pools[4], pools[5],
      batch2d, bng, bnb, w1, bl1, w2, bl2, w3p, bl3p)


def kernel(x, edge_index, batch, W_rel1, W_root1, b1, W_rel2, W_root2, b2,
           ln_g, ln_b, bn_g, bn_b, W_l1, b_l1, W_l2, b_l2, W_l3, b_l3):
    src4 = edge_index[0].reshape(NS, NBLK, IBLK, EK)
    dst4 = edge_index[1].reshape(NS, NBLK, IBLK, EK)
    zeros_nh = jnp.zeros((N, HD), jnp.float32)
    wcat1 = jnp.concatenate([W_rel1, W_root1], axis=1)
    wcat2 = jnp.concatenate([W_rel2, W_root2], axis=1)
    b1_2d = b1.reshape(1, D)
    b2_2d = b2.reshape(1, D)
    lng = ln_g.reshape(1, D)
    lnb = ln_b.reshape(1, D)

    wcat1 = wcat1.astype(jnp.bfloat16)
    wcat2 = wcat2.astype(jnp.bfloat16)

    yrel, z = _matmul(x.astype(jnp.bfloat16), wcat1, b1_2d)
    agg = _edge_kernel(yrel.reshape(NC * N, HD), src4, dst4, zeros_nh)
    h2s = []
    for _ in range(2):
        h2, yrel, z = _ln_matmul(agg, z, lng, lnb, wcat2, b2_2d)
        h2s.append(h2)
        agg = _edge_kernel(yrel.reshape(NC * N, HD), src4, dst4, zeros_nh)
    _, h2 = _layernorm(agg, z, lng, lnb)
    h2s.append(h2)

    pools = []
    for h2 in h2s:
        ps, px = _pool_kernel(h2, batch)
        pools += [ps.reshape(NC, NG, HD), px.reshape(NC, NG, HD)]

    w3p = jnp.pad(W_l3, ((0, 0), (0, HD - NCLS)))
    bl3p = jnp.pad(b_l3, (0, HD - NCLS)).reshape(1, HD)
    out128 = _head(pools, batch.reshape(1, N), bn_g.reshape(1, -1),
                   bn_b.reshape(1, -1), W_l1, b_l1.reshape(1, -1),
                   W_l2, b_l2.reshape(1, -1), w3p, bl3p)
    return out128[:, :NCLS]


# speedup vs baseline: 6.4934x; 1.0222x over previous
import functools

import numpy as np

import jax
import jax.numpy as jnp
from jax import lax
from jax.experimental import pallas as pl
from jax.experimental.pallas import tpu as pltpu
from jax.experimental.pallas import tpu_sc as plsc

N = 10000
E = 160000
D = 256
HD = 128
NG = 64
NCLS = 2
NC, NS = 2, 16
EPS = 1e-5
NEG = -3.4e38

EK = 80
EPT = E // NS
NCHUNK = EPT // EK
IBLK = 25
NBLK = NCHUNK // IBLK
RPT = 640
RPT_LAST = N - (NS - 1) * RPT
GPT = 8

BLK = 1000

_GDN = lax.GatherDimensionNumbers(
    offset_dims=(), collapsed_slice_dims=(0,), start_index_map=(0,))


def _splat_lane(vec, r):
    idx = jnp.full((16, 1), r, jnp.int32)
    return lax.gather(vec, idx, dimension_numbers=_GDN, slice_sizes=(1,),
                      mode=lax.GatherScatterMode.PROMISE_IN_BOUNDS)


def _mm_body(h_ref, w_ref, b_ref, yrel_ref, z_ref):
    y = jnp.dot(h_ref[...], w_ref[...], preferred_element_type=jnp.float32)
    yrel_ref[0] = y[:, :HD]
    yrel_ref[1] = y[:, HD:D]
    z_ref[...] = y[:, D:] + b_ref[...]


def _matmul(h, wcat, b2d):
    return pl.pallas_call(
        _mm_body,
        grid=(N // BLK,),
        in_specs=[
            pl.BlockSpec((BLK, D), lambda i: (i, 0)),
            pl.BlockSpec((D, 2 * D), lambda i: (0, 0)),
            pl.BlockSpec((1, D), lambda i: (0, 0)),
        ],
        out_specs=[
            pl.BlockSpec((NC, BLK, HD), lambda i: (0, i, 0)),
            pl.BlockSpec((BLK, D), lambda i: (i, 0)),
        ],
        out_shape=[
            jax.ShapeDtypeStruct((NC, N, HD), jnp.float32),
            jax.ShapeDtypeStruct((N, D), jnp.float32),
        ],
    )(h, wcat, b2d)


def _ln_body(agg_ref, z_ref, g_ref, b_ref, h_ref, h2_ref):
    a = jnp.concatenate([agg_ref[0], agg_ref[1]], axis=1) + z_ref[...]
    m = jnp.mean(a, axis=1, keepdims=True)
    v = jnp.mean((a - m) ** 2, axis=1, keepdims=True)
    hn = (a - m) * lax.rsqrt(v + EPS) * g_ref[...] + b_ref[...]
    h_ref[...] = hn.astype(jnp.bfloat16)
    h2_ref[0] = hn[:, :HD]
    h2_ref[1] = hn[:, HD:]


def _layernorm(agg, z, g2d, b2d):
    return pl.pallas_call(
        _ln_body,
        grid=(N // BLK,),
        in_specs=[
            pl.BlockSpec((NC, BLK, HD), lambda i: (0, i, 0)),
            pl.BlockSpec((BLK, D), lambda i: (i, 0)),
            pl.BlockSpec((1, D), lambda i: (0, 0)),
            pl.BlockSpec((1, D), lambda i: (0, 0)),
        ],
        out_specs=[
            pl.BlockSpec((BLK, D), lambda i: (i, 0)),
            pl.BlockSpec((NC, BLK, HD), lambda i: (0, i, 0)),
        ],
        out_shape=[
            jax.ShapeDtypeStruct((N, D), jnp.bfloat16),
            jax.ShapeDtypeStruct((NC, N, HD), jnp.float32),
        ],
    )(agg, z, g2d, b2d)


def _lnmm_body(agg_ref, z_ref, g_ref, b_ref, w_ref, bn_ref,
               h2_ref, yrel_ref, zn_ref):
    a = jnp.concatenate([agg_ref[0], agg_ref[1]], axis=1) + z_ref[...]
    m = jnp.mean(a, axis=1, keepdims=True)
    v = jnp.mean((a - m) ** 2, axis=1, keepdims=True)
    hn = (a - m) * lax.rsqrt(v + EPS) * g_ref[...] + b_ref[...]
    h2_ref[0] = hn[:, :HD]
    h2_ref[1] = hn[:, HD:]
    y = jnp.dot(hn.astype(jnp.bfloat16), w_ref[...],
                preferred_element_type=jnp.float32)
    yrel_ref[0] = y[:, :HD]
    yrel_ref[1] = y[:, HD:D]
    zn_ref[...] = y[:, D:] + bn_ref[...]


def _ln_matmul(agg, z, g2d, b2d, wcat, bn2d):
    return pl.pallas_call(
        _lnmm_body,
        grid=(N // BLK,),
        in_specs=[
            pl.BlockSpec((NC, BLK, HD), lambda i: (0, i, 0)),
            pl.BlockSpec((BLK, D), lambda i: (i, 0)),
            pl.BlockSpec((1, D), lambda i: (0, 0)),
            pl.BlockSpec((1, D), lambda i: (0, 0)),
            pl.BlockSpec((D, 2 * D), lambda i: (0, 0)),
            pl.BlockSpec((1, D), lambda i: (0, 0)),
        ],
        out_specs=[
            pl.BlockSpec((NC, BLK, HD), lambda i: (0, i, 0)),
            pl.BlockSpec((NC, BLK, HD), lambda i: (0, i, 0)),
            pl.BlockSpec((BLK, D), lambda i: (i, 0)),
        ],
        out_shape=[
            jax.ShapeDtypeStruct((NC, N, HD), jnp.float32),
            jax.ShapeDtypeStruct((NC, N, HD), jnp.float32),
            jax.ShapeDtypeStruct((N, D), jnp.float32),
        ],
    )(agg, z, g2d, b2d, wcat, bn2d)


_SC_MESH = plsc.VectorSubcoreMesh(core_axis_name="c", subcore_axis_name="s")


@functools.partial(
    pl.kernel,
    out_type=jax.ShapeDtypeStruct((NC, N, HD), jnp.float32),
    mesh=_SC_MESH,
    compiler_params=pltpu.CompilerParams(needs_layout_passes=False),
    scratch_types=[
        pltpu.VMEM((IBLK, EK), jnp.int32),
        pltpu.VMEM((IBLK, EK), jnp.int32),
        pltpu.VMEM((3, EK, HD), jnp.float32),
        pltpu.VMEM_SHARED((N, HD), jnp.float32),
        pltpu.SemaphoreType.DMA((3,)),
        pltpu.SemaphoreType.DMA((3,)),
    ],
)
def _edge_kernel(ytab, src4, dst4, zeros, agg_out, src_v, dst_v, rows_v, acc, sem_g, sem_s):
    c = lax.axis_index("c")
    s = lax.axis_index("s")
    @pl.when(s < NS - 1)
    def _():
        pltpu.sync_copy(zeros.at[pl.ds(s * RPT, RPT)], acc.at[pl.ds(s * RPT, RPT)])

    @pl.when(s == NS - 1)
    def _():
        pltpu.sync_copy(zeros.at[pl.ds(s * RPT, RPT_LAST)],
                        acc.at[pl.ds(s * RPT, RPT_LAST)])

    off = c * N
    plsc.subcore_barrier()

    def _block(blk, _):
        pltpu.sync_copy(src4.at[s, blk], src_v)
        pltpu.sync_copy(dst4.at[s, blk], dst_v)

        def _add_off(i, _):
            for j in range(EK // 16):
                src_v[i, pl.ds(j * 16, 16)] = src_v[i, pl.ds(j * 16, 16)] + off
            return 0

        lax.fori_loop(0, IBLK, _add_off, 0)

        pltpu.async_copy(ytab.at[src_v.at[0]], rows_v.at[0], sem_g.at[0])
        pltpu.async_copy(ytab.at[src_v.at[1]], rows_v.at[1], sem_g.at[1])

        def _chunk(i, _):
            sl = lax.rem(i, 3)
            pltpu.make_async_copy(ytab.at[src_v.at[i]], rows_v.at[sl],
                                  sem_g.at[sl]).wait()
            pltpu.async_copy(rows_v.at[sl], acc.at[dst_v.at[i]],
                             sem_s.at[sl], add=True)

            @pl.when(i + 2 < IBLK)
            def _():
                sl2 = lax.rem(i + 2, 3)

                @pl.when(i >= 1)
                def _():
                    pltpu.make_async_copy(rows_v.at[sl2], acc.at[dst_v.at[0]],
                                          sem_s.at[sl2]).wait()

                pltpu.async_copy(ytab.at[src_v.at[i + 2]], rows_v.at[sl2],
                                 sem_g.at[sl2])
            return 0

        lax.fori_loop(0, IBLK, _chunk, 0)
        for k in range(3):
            pltpu.make_async_copy(rows_v.at[k], acc.at[dst_v.at[0]],
                                  sem_s.at[k]).wait()
        return 0

    lax.fori_loop(0, NBLK, _block, 0)
    plsc.subcore_barrier()

    @pl.when(s < NS - 1)
    def _():
        pltpu.sync_copy(acc.at[pl.ds(s * RPT, RPT)],
                        agg_out.at[c, pl.ds(s * RPT, RPT)])

    @pl.when(s == NS - 1)
    def _():
        pltpu.sync_copy(acc.at[pl.ds(s * RPT, RPT_LAST)],
                        agg_out.at[c, pl.ds(s * RPT, RPT_LAST)])


@functools.partial(
    pl.kernel,
    out_type=tuple(
        jax.ShapeDtypeStruct((NC, NG * HD), jnp.float32) for _ in range(6)),
    mesh=_SC_MESH,
    compiler_params=pltpu.CompilerParams(needs_layout_passes=False),
    scratch_types=[
        pltpu.VMEM((RPT // 2, HD), jnp.float32),
        pltpu.VMEM((RPT,), jnp.int32),
        pltpu.VMEM((NG * HD,), jnp.float32),
        pltpu.VMEM((NG * HD,), jnp.float32),
        pltpu.VMEM((NS, GPT * HD), jnp.float32),
        pltpu.VMEM((GPT * HD,), jnp.float32),
        pltpu.VMEM_SHARED((NS, NG * HD), jnp.float32),
    ],
)
def _pool_kernel(h2a, h2b, h2c, batch, ps1, px1, ps2, px2, ps3, px3,
                 rows_v, bid_v, sum_v, max_v, tmp_v, res_v, stage):
    c = lax.axis_index("c")
    s = lax.axis_index("s")
    base = s * RPT
    iota = lax.iota(jnp.int32, 16)
    z16 = jnp.zeros((16,), jnp.float32)
    neg16 = jnp.full((16,), NEG, jnp.float32)
    seg = GPT * HD
    RPB = RPT // 2

    @pl.when(s < NS - 1)
    def _():
        pltpu.sync_copy(batch.at[pl.ds(base, RPT)], bid_v)

    @pl.when(s == NS - 1)
    def _():
        pltpu.sync_copy(batch.at[pl.ds(base, RPT_LAST)], bid_v.at[pl.ds(0, RPT_LAST)])

    for h2, psum, pmax in ((h2a, ps1, px1), (h2b, ps2, px2), (h2c, ps3, px3)):
        plsc.subcore_barrier()

        def _init(i, _):
            sum_v[pl.ds(i * 16, 16)] = z16
            max_v[pl.ds(i * 16, 16)] = neg16
            return 0

        lax.fori_loop(0, NG * HD // 16, _init, 0)

        def _group_at(poff, gi):
            gbase = gi * 16
            bvec = bid_v[pl.ds(poff + gbase, 16)]
            same = jnp.max(bvec) == jnp.min(bvec)

            @pl.when(same)
            def _():
                rowbase = _splat_lane(bvec, 0) * HD
                for j in range(HD // 16):
                    acc_s = rows_v[gbase, pl.ds(j * 16, 16)]
                    acc_m = acc_s
                    for r in range(1, 16):
                        v = rows_v[gbase + r, pl.ds(j * 16, 16)]
                        acc_s = acc_s + v
                        acc_m = jnp.maximum(acc_m, v)
                    fi = rowbase + (j * 16 + iota)
                    cs = plsc.load_gather(sum_v, [fi])
                    plsc.store_scatter(sum_v, [fi], cs + acc_s)
                    cm = plsc.load_gather(max_v, [fi])
                    plsc.store_scatter(max_v, [fi], jnp.maximum(cm, acc_m))

            @pl.when(jnp.logical_not(same))
            def _():
                def _row(r, _):
                    rowbase = _splat_lane(bvec, r) * HD
                    for j in range(HD // 16):
                        fi = rowbase + (j * 16 + iota)
                        v = rows_v[gbase + r, pl.ds(j * 16, 16)]
                        cs = plsc.load_gather(sum_v, [fi])
                        plsc.store_scatter(sum_v, [fi], cs + v)
                        cm = plsc.load_gather(max_v, [fi])
                        plsc.store_scatter(max_v, [fi], jnp.maximum(cm, v))
                    return 0

                lax.fori_loop(0, 16, _row, 0)

        pltpu.sync_copy(h2.at[c, pl.ds(base, RPB)], rows_v)

        def _g0(gi, _):
            _group_at(0, gi)
            return 0

        lax.fori_loop(0, RPB // 16, _g0, 0)

        @pl.when(s < NS - 1)
        def _():
            pltpu.sync_copy(h2.at[c, pl.ds(base + RPB, RPB)], rows_v)

        @pl.when(s == NS - 1)
        def _():
            pltpu.sync_copy(h2.at[c, pl.ds(base + RPB, RPT_LAST - RPB)],
                            rows_v.at[pl.ds(0, RPT_LAST - RPB)])

        ng1 = jnp.where(s < NS - 1, RPB // 16, (RPT_LAST - RPB) // 16)

        def _g1(gi, _):
            _group_at(RPB, gi)
            return 0

        lax.fori_loop(0, ng1, _g1, 0)

        pltpu.sync_copy(sum_v, stage.at[s])
        plsc.subcore_barrier()

        @pl.when(s < NG // GPT)
        def _():
            pltpu.sync_copy(stage.at[pl.ds(0, NS), pl.ds(seg * s, seg)], tmp_v)

            def _zero(j, _):
                res_v[pl.ds(j * 16, 16)] = z16
                return 0

            lax.fori_loop(0, seg // 16, _zero, 0)

            def _msum(t, _):
                for j in range(seg // 16):
                    res_v[pl.ds(j * 16, 16)] = (
                        res_v[pl.ds(j * 16, 16)] + tmp_v[t, pl.ds(j * 16, 16)])
                return 0

            lax.fori_loop(0, NS, _msum, 0)
            pltpu.sync_copy(res_v, psum.at[c, pl.ds(seg * s, seg)])

        plsc.subcore_barrier()
        pltpu.sync_copy(max_v, stage.at[s])
        plsc.subcore_barrier()

        @pl.when(s < NG // GPT)
        def _():
            pltpu.sync_copy(stage.at[pl.ds(0, NS), pl.ds(seg * s, seg)], tmp_v)

            def _zneg(j, _):
                res_v[pl.ds(j * 16, 16)] = neg16
                return 0

            lax.fori_loop(0, seg // 16, _zneg, 0)

            def _mmax(t, _):
                for j in range(seg // 16):
                    res_v[pl.ds(j * 16, 16)] = jnp.maximum(
                        res_v[pl.ds(j * 16, 16)], tmp_v[t, pl.ds(j * 16, 16)])
                return 0

            lax.fori_loop(0, NS, _mmax, 0)
            pltpu.sync_copy(res_v, pmax.at[c, pl.ds(seg * s, seg)])


def _head_body(ps1, px1, ps2, px2, ps3, px3, bt, bng, bnb,
               w1, bl1, w2, bl2, w3, bl3, out_ref):
    gids = lax.broadcasted_iota(jnp.int32, (NG, 1), 0)
    cnt = jnp.sum((bt[...] == gids).astype(jnp.float32), axis=1, keepdims=True)
    cnt_c = jnp.maximum(cnt, 1.0)
    pieces = []
    for ps, px in ((ps1, px1), (ps2, px2), (ps3, px3)):
        sm = jnp.concatenate([ps[0], ps[1]], axis=1)
        mx = jnp.concatenate([px[0], px[1]], axis=1)
        pieces += [sm / cnt_c, jnp.where(cnt > 0.0, mx, 0.0), sm]
    hk = jnp.concatenate(pieces, axis=1)
    bm = jnp.mean(hk, axis=0, keepdims=True)
    bv = jnp.mean((hk - bm) ** 2, axis=0, keepdims=True)
    xn = (hk - bm) * lax.rsqrt(bv + EPS) * bng[...] + bnb[...]
    x1 = jnp.maximum(jnp.dot(xn, w1[...], preferred_element_type=jnp.float32) + bl1[...], 0.0)
    x2 = jnp.maximum(jnp.dot(x1, w2[...], preferred_element_type=jnp.float32) + bl2[...], 0.0)
    lg = jnp.dot(x2, w3[...], preferred_element_type=jnp.float32) + bl3[...]
    valid = lax.broadcasted_iota(jnp.int32, (NG, HD), 1) < NCLS
    mxl = jnp.max(jnp.where(valid, lg, NEG), axis=1, keepdims=True)
    ex = jnp.where(valid, jnp.exp(lg - mxl), 0.0)
    lse = jnp.log(jnp.sum(ex, axis=1, keepdims=True)) + mxl
    out_ref[...] = lg - lse


def _head(pools, batch2d, bng, bnb, w1, bl1, w2, bl2, w3p, bl3p):
    return pl.pallas_call(
        _head_body,
        out_shape=jax.ShapeDtypeStruct((NG, HD), jnp.float32),
    )(pools[0], pools[1], pools[2], pools[3], pools[4], pools[5],
      batch2d, bng, bnb, w1, bl1, w2, bl2, w3p, bl3p)


def kernel(x, edge_index, batch, W_rel1, W_root1, b1, W_rel2, W_root2, b2,
           ln_g, ln_b, bn_g, bn_b, W_l1, b_l1, W_l2, b_l2, W_l3, b_l3):
    src4 = edge_index[0].reshape(NS, NBLK, IBLK, EK)
    dst4 = edge_index[1].reshape(NS, NBLK, IBLK, EK)
    zeros_nh = jnp.zeros((N, HD), jnp.float32)
    wcat1 = jnp.concatenate([W_rel1, W_root1], axis=1)
    wcat2 = jnp.concatenate([W_rel2, W_root2], axis=1)
    b1_2d = b1.reshape(1, D)
    b2_2d = b2.reshape(1, D)
    lng = ln_g.reshape(1, D)
    lnb = ln_b.reshape(1, D)

    wcat1 = wcat1.astype(jnp.bfloat16)
    wcat2 = wcat2.astype(jnp.bfloat16)

    yrel, z = _matmul(x.astype(jnp.bfloat16), wcat1, b1_2d)
    agg = _edge_kernel(yrel.reshape(NC * N, HD), src4, dst4, zeros_nh)
    h2s = []
    for _ in range(2):
        h2, yrel, z = _ln_matmul(agg, z, lng, lnb, wcat2, b2_2d)
        h2s.append(h2)
        agg = _edge_kernel(yrel.reshape(NC * N, HD), src4, dst4, zeros_nh)
    _, h2 = _layernorm(agg, z, lng, lnb)
    h2s.append(h2)

    pools = [p.reshape(NC, NG, HD)
             for p in _pool_kernel(h2s[0], h2s[1], h2s[2], batch)]

    w3p = jnp.pad(W_l3, ((0, 0), (0, HD - NCLS)))
    bl3p = jnp.pad(b_l3, (0, HD - NCLS)).reshape(1, HD)
    out128 = _head(pools, batch.reshape(1, N), bn_g.reshape(1, -1),
                   bn_b.reshape(1, -1), W_l1, b_l1.reshape(1, -1),
                   W_l2, b_l2.reshape(1, -1), w3p, bl3p)
    return out128[:, :NCLS]


# speedup vs baseline: 6.8761x; 1.0589x over previous
import functools

import numpy as np

import jax
import jax.numpy as jnp
from jax import lax
from jax.experimental import pallas as pl
from jax.experimental.pallas import tpu as pltpu
from jax.experimental.pallas import tpu_sc as plsc

N = 10000
E = 160000
D = 256
HD = 128
NG = 64
NCLS = 2
NC, NS = 2, 16
EPS = 1e-5
NEG = -3.4e38

EK = 80
EPT = E // NS
NCHUNK = EPT // EK
IBLK = 25
NBLK = NCHUNK // IBLK
RPT = 640
RPT_LAST = N - (NS - 1) * RPT
GPT = 8

BLK = 1000

_GDN = lax.GatherDimensionNumbers(
    offset_dims=(), collapsed_slice_dims=(0,), start_index_map=(0,))


def _splat_lane(vec, r):
    idx = jnp.full((16, 1), r, jnp.int32)
    return lax.gather(vec, idx, dimension_numbers=_GDN, slice_sizes=(1,),
                      mode=lax.GatherScatterMode.PROMISE_IN_BOUNDS)


def _mm_body(h_ref, w_ref, b_ref, yrel_ref, z_ref):
    y = jnp.dot(h_ref[...], w_ref[...], preferred_element_type=jnp.float32)
    yrel_ref[0] = y[:, :HD]
    yrel_ref[1] = y[:, HD:D]
    z_ref[...] = y[:, D:] + b_ref[...]


def _matmul(h, wcat, b2d):
    return pl.pallas_call(
        _mm_body,
        grid=(N // BLK,),
        in_specs=[
            pl.BlockSpec((BLK, D), lambda i: (i, 0)),
            pl.BlockSpec((D, 2 * D), lambda i: (0, 0)),
            pl.BlockSpec((1, D), lambda i: (0, 0)),
        ],
        out_specs=[
            pl.BlockSpec((NC, BLK, HD), lambda i: (0, i, 0)),
            pl.BlockSpec((BLK, D), lambda i: (i, 0)),
        ],
        out_shape=[
            jax.ShapeDtypeStruct((NC, N, HD), jnp.float32),
            jax.ShapeDtypeStruct((N, D), jnp.float32),
        ],
    )(h, wcat, b2d)


def _ln_body(agg_ref, z_ref, g_ref, b_ref, h_ref, h2_ref):
    a = jnp.concatenate([agg_ref[0], agg_ref[1]], axis=1) + z_ref[...]
    m = jnp.mean(a, axis=1, keepdims=True)
    v = jnp.mean((a - m) ** 2, axis=1, keepdims=True)
    hn = (a - m) * lax.rsqrt(v + EPS) * g_ref[...] + b_ref[...]
    h_ref[...] = hn.astype(jnp.bfloat16)
    h2_ref[0] = hn[:, :HD]
    h2_ref[1] = hn[:, HD:]


def _layernorm(agg, z, g2d, b2d):
    return pl.pallas_call(
        _ln_body,
        grid=(N // BLK,),
        in_specs=[
            pl.BlockSpec((NC, BLK, HD), lambda i: (0, i, 0)),
            pl.BlockSpec((BLK, D), lambda i: (i, 0)),
            pl.BlockSpec((1, D), lambda i: (0, 0)),
            pl.BlockSpec((1, D), lambda i: (0, 0)),
        ],
        out_specs=[
            pl.BlockSpec((BLK, D), lambda i: (i, 0)),
            pl.BlockSpec((NC, BLK, HD), lambda i: (0, i, 0)),
        ],
        out_shape=[
            jax.ShapeDtypeStruct((N, D), jnp.bfloat16),
            jax.ShapeDtypeStruct((NC, N, HD), jnp.float32),
        ],
    )(agg, z, g2d, b2d)


def _lnmm_body(agg_ref, z_ref, g_ref, b_ref, w_ref, bn_ref,
               h2_ref, yrel_ref, zn_ref):
    a = jnp.concatenate([agg_ref[0], agg_ref[1]], axis=1) + z_ref[...]
    m = jnp.mean(a, axis=1, keepdims=True)
    v = jnp.mean((a - m) ** 2, axis=1, keepdims=True)
    hn = (a - m) * lax.rsqrt(v + EPS) * g_ref[...] + b_ref[...]
    h2_ref[0] = hn[:, :HD]
    h2_ref[1] = hn[:, HD:]
    y = jnp.dot(hn.astype(jnp.bfloat16), w_ref[...],
                preferred_element_type=jnp.float32)
    yrel_ref[0] = y[:, :HD]
    yrel_ref[1] = y[:, HD:D]
    zn_ref[...] = y[:, D:] + bn_ref[...]


def _ln_matmul(agg, z, g2d, b2d, wcat, bn2d):
    return pl.pallas_call(
        _lnmm_body,
        grid=(N // BLK,),
        in_specs=[
            pl.BlockSpec((NC, BLK, HD), lambda i: (0, i, 0)),
            pl.BlockSpec((BLK, D), lambda i: (i, 0)),
            pl.BlockSpec((1, D), lambda i: (0, 0)),
            pl.BlockSpec((1, D), lambda i: (0, 0)),
            pl.BlockSpec((D, 2 * D), lambda i: (0, 0)),
            pl.BlockSpec((1, D), lambda i: (0, 0)),
        ],
        out_specs=[
            pl.BlockSpec((NC, BLK, HD), lambda i: (0, i, 0)),
            pl.BlockSpec((NC, BLK, HD), lambda i: (0, i, 0)),
            pl.BlockSpec((BLK, D), lambda i: (i, 0)),
        ],
        out_shape=[
            jax.ShapeDtypeStruct((NC, N, HD), jnp.float32),
            jax.ShapeDtypeStruct((NC, N, HD), jnp.float32),
            jax.ShapeDtypeStruct((N, D), jnp.float32),
        ],
    )(agg, z, g2d, b2d, wcat, bn2d)


_SC_MESH = plsc.VectorSubcoreMesh(core_axis_name="c", subcore_axis_name="s")


@functools.partial(
    pl.kernel,
    out_type=jax.ShapeDtypeStruct((NC, N, HD), jnp.float32),
    mesh=_SC_MESH,
    compiler_params=pltpu.CompilerParams(needs_layout_passes=False),
    scratch_types=[
        pltpu.VMEM((IBLK, EK), jnp.int32),
        pltpu.VMEM((IBLK, EK), jnp.int32),
        pltpu.VMEM((3, EK, HD), jnp.float32),
        pltpu.VMEM_SHARED((N, HD), jnp.float32),
        pltpu.SemaphoreType.DMA((3,)),
        pltpu.SemaphoreType.DMA((3,)),
    ],
)
def _edge_kernel(ytab, src4, dst4, zeros, agg_out, src_v, dst_v, rows_v, acc, sem_g, sem_s):
    c = lax.axis_index("c")
    s = lax.axis_index("s")
    @pl.when(s < NS - 1)
    def _():
        pltpu.sync_copy(zeros.at[pl.ds(s * RPT, RPT)], acc.at[pl.ds(s * RPT, RPT)])

    @pl.when(s == NS - 1)
    def _():
        pltpu.sync_copy(zeros.at[pl.ds(s * RPT, RPT_LAST)],
                        acc.at[pl.ds(s * RPT, RPT_LAST)])

    off = c * N
    plsc.subcore_barrier()

    def _block(blk, _):
        pltpu.sync_copy(src4.at[s, blk], src_v)
        pltpu.sync_copy(dst4.at[s, blk], dst_v)

        def _add_off(i, _):
            for j in range(EK // 16):
                src_v[i, pl.ds(j * 16, 16)] = src_v[i, pl.ds(j * 16, 16)] + off
            return 0

        lax.fori_loop(0, IBLK, _add_off, 0)

        pltpu.async_copy(ytab.at[src_v.at[0]], rows_v.at[0], sem_g.at[0])
        pltpu.async_copy(ytab.at[src_v.at[1]], rows_v.at[1], sem_g.at[1])

        def _chunk(i, _):
            sl = lax.rem(i, 3)
            pltpu.make_async_copy(ytab.at[src_v.at[i]], rows_v.at[sl],
                                  sem_g.at[sl]).wait()
            pltpu.async_copy(rows_v.at[sl], acc.at[dst_v.at[i]],
                             sem_s.at[sl], add=True)

            @pl.when(i + 2 < IBLK)
            def _():
                sl2 = lax.rem(i + 2, 3)

                @pl.when(i >= 1)
                def _():
                    pltpu.make_async_copy(rows_v.at[sl2], acc.at[dst_v.at[0]],
                                          sem_s.at[sl2]).wait()

                pltpu.async_copy(ytab.at[src_v.at[i + 2]], rows_v.at[sl2],
                                 sem_g.at[sl2])
            return 0

        lax.fori_loop(0, IBLK, _chunk, 0)
        for k in range(3):
            pltpu.make_async_copy(rows_v.at[k], acc.at[dst_v.at[0]],
                                  sem_s.at[k]).wait()
        return 0

    lax.fori_loop(0, NBLK, _block, 0)
    plsc.subcore_barrier()

    @pl.when(s < NS - 1)
    def _():
        pltpu.sync_copy(acc.at[pl.ds(s * RPT, RPT)],
                        agg_out.at[c, pl.ds(s * RPT, RPT)])

    @pl.when(s == NS - 1)
    def _():
        pltpu.sync_copy(acc.at[pl.ds(s * RPT, RPT_LAST)],
                        agg_out.at[c, pl.ds(s * RPT, RPT_LAST)])


@functools.partial(
    pl.kernel,
    out_type=tuple(
        jax.ShapeDtypeStruct((NC, NG, HD), jnp.float32) if i % 2 == 0
        else jax.ShapeDtypeStruct((NC, NG * HD), jnp.float32)
        for i in range(6)),
    mesh=_SC_MESH,
    compiler_params=pltpu.CompilerParams(needs_layout_passes=False),
    scratch_types=[
        pltpu.VMEM((128, HD), jnp.float32),
        pltpu.VMEM((5, 128), jnp.int32),
        pltpu.VMEM((16,), jnp.int32),
        pltpu.VMEM((NG * HD,), jnp.float32),
        pltpu.VMEM((NS, GPT * HD), jnp.float32),
        pltpu.VMEM((GPT * HD,), jnp.float32),
        pltpu.VMEM_SHARED((NG, HD), jnp.float32),
        pltpu.VMEM_SHARED((NS, NG * HD), jnp.float32),
        pltpu.SemaphoreType.DMA,
    ],
)
def _pool_kernel(h2a, h2b, h2c, batch, zeros, ps1, px1, ps2, px2, ps3, px3,
                 rows_v, bid2_v, bid16_v, max_v, tmp_v, res_v, sumacc, stage, sem):
    c = lax.axis_index("c")
    s = lax.axis_index("s")
    base = s * RPT
    iota = lax.iota(jnp.int32, 16)
    z16 = jnp.zeros((16,), jnp.float32)
    neg16 = jnp.full((16,), NEG, jnp.float32)
    seg = GPT * HD
    nfull = jnp.where(s < NS - 1, 5, 3)

    @pl.when(s < NS - 1)
    def _():
        for k in range(5):
            pltpu.sync_copy(batch.at[pl.ds(base + k * 128, 128)], bid2_v.at[k])

    @pl.when(s == NS - 1)
    def _():
        for k in range(3):
            pltpu.sync_copy(batch.at[pl.ds(base + k * 128, 128)], bid2_v.at[k])
        pltpu.sync_copy(batch.at[pl.ds(base + 384, 16)], bid16_v)
        pltpu.sync_copy(batch.at[pl.ds(base + 384, 16)], bid2_v.at[3, pl.ds(0, 16)])

    def _max_group(k, g, roff):
        bvec = bid2_v[k, pl.ds(g * 16, 16)]
        same = jnp.max(bvec) == jnp.min(bvec)

        @pl.when(same)
        def _():
            rowbase = _splat_lane(bvec, 0) * HD
            for j in range(HD // 16):
                acc_m = rows_v[roff, pl.ds(j * 16, 16)]
                for r in range(1, 16):
                    acc_m = jnp.maximum(acc_m, rows_v[roff + r, pl.ds(j * 16, 16)])
                fi = rowbase + (j * 16 + iota)
                cm = plsc.load_gather(max_v, [fi])
                plsc.store_scatter(max_v, [fi], jnp.maximum(cm, acc_m))

        @pl.when(jnp.logical_not(same))
        def _():
            def _row(r, _):
                rowbase = _splat_lane(bvec, r) * HD
                for j in range(HD // 16):
                    fi = rowbase + (j * 16 + iota)
                    cm = plsc.load_gather(max_v, [fi])
                    plsc.store_scatter(
                        max_v, [fi],
                        jnp.maximum(cm, rows_v[roff + r, pl.ds(j * 16, 16)]))
                return 0

            lax.fori_loop(0, 16, _row, 0)

    for h2, psum, pmax in ((h2a, ps1, px1), (h2b, ps2, px2), (h2c, ps3, px3)):
        plsc.subcore_barrier()

        @pl.when(s == 0)
        def _():
            pltpu.sync_copy(zeros.at[pl.ds(0, NG)], sumacc)

        def _init(i, _):
            max_v[pl.ds(i * 16, 16)] = neg16
            return 0

        lax.fori_loop(0, NG * HD // 16, _init, 0)
        plsc.subcore_barrier()

        def _chunk(k, _):
            pltpu.sync_copy(h2.at[c, pl.ds(base + k * 128, 128)], rows_v)
            pltpu.async_copy(rows_v, sumacc.at[bid2_v.at[k]], sem, add=True)

            def _g(g, _):
                _max_group(k, g, g * 16)
                return 0

            lax.fori_loop(0, 8, _g, 0)
            pltpu.make_async_copy(rows_v, sumacc.at[bid2_v.at[k]], sem).wait()
            return 0

        lax.fori_loop(0, nfull, _chunk, 0)

        @pl.when(s == NS - 1)
        def _():
            pltpu.sync_copy(h2.at[c, pl.ds(base + 384, 16)],
                            rows_v.at[pl.ds(0, 16)])
            pltpu.sync_copy(rows_v.at[pl.ds(0, 16)], sumacc.at[bid16_v],
                            add=True)
            _max_group(3, 0, 0)

        plsc.subcore_barrier()

        @pl.when(s < NG // GPT)
        def _():
            pltpu.sync_copy(sumacc.at[pl.ds(GPT * s, GPT)],
                            psum.at[c, pl.ds(GPT * s, GPT)])

        pltpu.sync_copy(max_v, stage.at[s])
        plsc.subcore_barrier()

        @pl.when(s < NG // GPT)
        def _():
            pltpu.sync_copy(stage.at[pl.ds(0, NS), pl.ds(seg * s, seg)], tmp_v)

            def _zneg(j, _):
                res_v[pl.ds(j * 16, 16)] = neg16
                return 0

            lax.fori_loop(0, seg // 16, _zneg, 0)

            def _mmax(t, _):
                for j in range(seg // 16):
                    res_v[pl.ds(j * 16, 16)] = jnp.maximum(
                        res_v[pl.ds(j * 16, 16)], tmp_v[t, pl.ds(j * 16, 16)])
                return 0

            lax.fori_loop(0, NS, _mmax, 0)
            pltpu.sync_copy(res_v, pmax.at[c, pl.ds(seg * s, seg)])


def _head_body(ps1, px1, ps2, px2, ps3, px3, bt, bng, bnb,
               w1, bl1, w2, bl2, w3, bl3, out_ref):
    gids = lax.broadcasted_iota(jnp.int32, (NG, 1), 0)
    cnt = jnp.sum((bt[...] == gids).astype(jnp.float32), axis=1, keepdims=True)
    cnt_c = jnp.maximum(cnt, 1.0)
    pieces = []
    for ps, px in ((ps1, px1), (ps2, px2), (ps3, px3)):
        sm = jnp.concatenate([ps[0], ps[1]], axis=1)
        mx = jnp.concatenate([px[0], px[1]], axis=1)
        pieces += [sm / cnt_c, jnp.where(cnt > 0.0, mx, 0.0), sm]
    hk = jnp.concatenate(pieces, axis=1)
    bm = jnp.mean(hk, axis=0, keepdims=True)
    bv = jnp.mean((hk - bm) ** 2, axis=0, keepdims=True)
    xn = (hk - bm) * lax.rsqrt(bv + EPS) * bng[...] + bnb[...]
    x1 = jnp.maximum(jnp.dot(xn, w1[...], preferred_element_type=jnp.float32) + bl1[...], 0.0)
    x2 = jnp.maximum(jnp.dot(x1, w2[...], preferred_element_type=jnp.float32) + bl2[...], 0.0)
    lg = jnp.dot(x2, w3[...], preferred_element_type=jnp.float32) + bl3[...]
    valid = lax.broadcasted_iota(jnp.int32, (NG, HD), 1) < NCLS
    mxl = jnp.max(jnp.where(valid, lg, NEG), axis=1, keepdims=True)
    ex = jnp.where(valid, jnp.exp(lg - mxl), 0.0)
    lse = jnp.log(jnp.sum(ex, axis=1, keepdims=True)) + mxl
    out_ref[...] = lg - lse


def _head(pools, batch2d, bng, bnb, w1, bl1, w2, bl2, w3p, bl3p):
    return pl.pallas_call(
        _head_body,
        out_shape=jax.ShapeDtypeStruct((NG, HD), jnp.float32),
    )(pools[0], pools[1], pools[2], pools[3], pools[4], pools[5],
      batch2d, bng, bnb, w1, bl1, w2, bl2, w3p, bl3p)


def kernel(x, edge_index, batch, W_rel1, W_root1, b1, W_rel2, W_root2, b2,
           ln_g, ln_b, bn_g, bn_b, W_l1, b_l1, W_l2, b_l2, W_l3, b_l3):
    src4 = edge_index[0].reshape(NS, NBLK, IBLK, EK)
    dst4 = edge_index[1].reshape(NS, NBLK, IBLK, EK)
    zeros_nh = jnp.zeros((N, HD), jnp.float32)
    wcat1 = jnp.concatenate([W_rel1, W_root1], axis=1)
    wcat2 = jnp.concatenate([W_rel2, W_root2], axis=1)
    b1_2d = b1.reshape(1, D)
    b2_2d = b2.reshape(1, D)
    lng = ln_g.reshape(1, D)
    lnb = ln_b.reshape(1, D)

    wcat1 = wcat1.astype(jnp.bfloat16)
    wcat2 = wcat2.astype(jnp.bfloat16)

    yrel, z = _matmul(x.astype(jnp.bfloat16), wcat1, b1_2d)
    agg = _edge_kernel(yrel.reshape(NC * N, HD), src4, dst4, zeros_nh)
    h2s = []
    for _ in range(2):
        h2, yrel, z = _ln_matmul(agg, z, lng, lnb, wcat2, b2_2d)
        h2s.append(h2)
        agg = _edge_kernel(yrel.reshape(NC * N, HD), src4, dst4, zeros_nh)
    _, h2 = _layernorm(agg, z, lng, lnb)
    h2s.append(h2)

    outs = _pool_kernel(h2s[0], h2s[1], h2s[2], batch, zeros_nh)
    pools = [p if i % 2 == 0 else p.reshape(NC, NG, HD)
             for i, p in enumerate(outs)]

    w3p = jnp.pad(W_l3, ((0, 0), (0, HD - NCLS)))
    bl3p = jnp.pad(b_l3, (0, HD - NCLS)).reshape(1, HD)
    out128 = _head(pools, batch.reshape(1, N), bn_g.reshape(1, -1),
                   bn_b.reshape(1, -1), W_l1, b_l1.reshape(1, -1),
                   W_l2, b_l2.reshape(1, -1), w3p, bl3p)
    return out128[:, :NCLS]
